# baseline (speedup 1.0000x reference)
import functools

import jax
import jax.numpy as jnp
from jax import lax
from jax.experimental import pallas as pl
from jax.experimental.pallas import tpu as pltpu
from jax.experimental.pallas import tpu_sc as plsc

N = 50000
E = 1600000
OUT = 16
HID = 32
LANE = 128

NV = 391
N_ROWS = 400
N_PAD = N_ROWS * LANE
E_ROWS = 12544
NC, NS = 2, 16
NW = NC * NS
TILE_E_ROWS = E_ROWS // NW
TILE_E_ROWS2 = E_ROWS // NS
NPT = N_PAD // NS
CH = 56
G = 4
NGRP = CH // G

_mesh = plsc.VectorSubcoreMesh(core_axis_name="c", subcore_axis_name="s")
_f32 = jnp.float32
_i32 = jnp.int32
_sc_params = pltpu.CompilerParams(use_tc_tiling_on_sc=False,
                                  needs_layout_passes=False)


def _rsqrt16(d):
    xi = plsc.bitcast(d, _i32)
    y = plsc.bitcast(jnp.int32(0x5F3759DF) - (xi >> 1), _f32)
    for _ in range(3):
        y = y * (1.5 - 0.5 * d * y * y)
    return y


def _tc_tables_body(ea, eb, W1, ta_o, tb_o):
    ta_o[...] = jnp.dot(ea[...], W1[0:16, :], preferred_element_type=_f32)
    tb_o[...] = jnp.dot(eb[...], W1[16:32, :], preferred_element_type=_f32)


def _tc_tables(emb_a, emb_b, W1):
    return pl.pallas_call(
        _tc_tables_body,
        out_shape=[
            jax.ShapeDtypeStruct((1000, HID), _f32),
            jax.ShapeDtypeStruct((1000, HID), _f32),
        ],
    )(emb_a, emb_b, W1)


RPT = N_ROWS // 16


def _sc_deg_body(dst_hbm, z2_hbm, deg_hbm, idx_v, ridx_v, ld_v, deg_sp):
    c = lax.axis_index("c")
    s = lax.axis_index("s")
    ones16 = jnp.ones((16,), _f32)
    zeros16 = jnp.zeros((16,), _f32)
    iota = lax.iota(_i32, 16)

    @pl.loop(0, N_ROWS)
    def _z(r):
        for u in range(LANE // 16):
            ld_v[r, pl.ds(u * 16, 16)] = zeros16

    for k in range(RPT):
        ridx_v[k, :] = iota + k * 16

    pltpu.sync_copy(z2_hbm, deg_sp.at[pl.ds(s * RPT, RPT)])
    plsc.subcore_barrier()

    base = (c * NS + s) * TILE_E_ROWS

    @pl.loop(0, TILE_E_ROWS // CH)
    def _deg_chunk(k):
        pltpu.sync_copy(dst_hbm.at[pl.ds(base + k * CH, CH)], idx_v)

        @pl.loop(0, CH)
        def _deg_row(j):
            for u in range(LANE // 16):
                d16 = idx_v[j, pl.ds(u * 16, 16)]
                plsc.addupdate_scatter(ld_v, [d16 >> 7, d16 & 127], ones16)

    @pl.loop(0, RPT)
    def _merge(k):
        pltpu.sync_copy(ld_v.at[pl.ds(k * 16, 16)],
                        deg_sp.at[ridx_v.at[k]], add=True)

    plsc.subcore_barrier()
    pltpu.sync_copy(deg_sp.at[pl.ds(s * RPT, RPT)],
                    deg_hbm.at[c, pl.ds(s * RPT, RPT)])


def _sc_deg(dst_r):
    z2 = jnp.zeros((RPT, LANE), _f32)
    return pl.kernel(
        _sc_deg_body,
        out_type=jax.ShapeDtypeStruct((NC, N_ROWS, LANE), _f32),
        mesh=_mesh,
        compiler_params=_sc_params,
        scratch_types=[
            pltpu.VMEM((CH, LANE), _i32),
            pltpu.VMEM((RPT, 16), _i32),
            pltpu.VMEM((N_ROWS, LANE), _f32),
            pltpu.VMEM_SHARED((N_ROWS, LANE), _f32),
        ],
    )(dst_r, z2)


def _sc_node1_body(x_hbm, ta_hbm, tb_hbm, W1_hbm, degp_hbm,
                   dis_hbm, g1a_hbm, g1b_hbm,
                   ta_v, tb_v, x_v, w1_v, deg_v, dis_v, ga_v, gb_v):
    c = lax.axis_index("c")
    s = lax.axis_index("s")
    w = s * NC + c

    pltpu.sync_copy(ta_hbm, ta_v)
    pltpu.sync_copy(tb_hbm, tb_v)
    pltpu.sync_copy(W1_hbm, w1_v)
    w1a = [w1_v[32 + k, pl.ds(0, 16)] for k in range(4)]
    w1b = [w1_v[32 + k, pl.ds(16, 16)] for k in range(4)]
    iota = lax.iota(_i32, 16)

    @pl.loop(w * NV // NW, (w + 1) * NV // NW)
    def _row(r):
        node0 = jnp.minimum(r * LANE, N - LANE)
        pltpu.sync_copy(x_hbm.at[pl.ds(node0, LANE)], x_v)
        pltpu.sync_copy(degp_hbm.at[0, pl.ds(node0, LANE)], deg_v.at[0])
        pltpu.sync_copy(degp_hbm.at[1, pl.ds(node0, LANE)], deg_v.at[1])
        for v in range(LANE // 16):
            sl = pl.ds(v * 16, 16)
            d = deg_v[0, sl] + deg_v[1, sl] + 1.0
            dis_v[sl] = _rsqrt16(d)
        pltpu.sync_copy(dis_v, dis_hbm.at[pl.ds(node0, LANE)])

        @pl.loop(0, LANE // 16)
        def _grp(t):
            base16 = t * 16
            nidx = iota + base16
            dis16 = dis_v[pl.ds(base16, 16)]
            ia16 = plsc.load_gather(
                x_v, [nidx, jnp.zeros((16,), _i32)]).astype(_i32)
            ib16 = plsc.load_gather(
                x_v, [nidx, jnp.full((16,), 1, _i32)]).astype(_i32)
            nums = [plsc.load_gather(x_v, [nidx, jnp.full((16,), 2 + k, _i32)])
                    for k in range(4)]
            for j in range(16):
                ian = ia16[j]
                ibn = ib16[j]
                ha = ta_v[ian, pl.ds(0, 16)] + tb_v[ibn, pl.ds(0, 16)]
                hb = ta_v[ian, pl.ds(16, 16)] + tb_v[ibn, pl.ds(16, 16)]
                for k in range(4):
                    nk = nums[k][j]
                    ha = ha + nk * w1a[k]
                    hb = hb + nk * w1b[k]
                dn = dis16[j]
                ga_v[base16 + j, :] = ha * dn
                gb_v[base16 + j, :] = hb * dn

        pltpu.sync_copy(ga_v, g1a_hbm.at[pl.ds(node0, LANE)])
        pltpu.sync_copy(gb_v, g1b_hbm.at[pl.ds(node0, LANE)])


def _sc_node1(x, ta, tb, W1, degp):
    return pl.kernel(
        _sc_node1_body,
        out_type=[
            jax.ShapeDtypeStruct((N_PAD,), _f32),
            jax.ShapeDtypeStruct((N_PAD, OUT), _f32),
            jax.ShapeDtypeStruct((N_PAD, OUT), _f32),
        ],
        mesh=_mesh,
        compiler_params=_sc_params,
        scratch_types=[
            pltpu.VMEM((1000, HID), _f32),
            pltpu.VMEM((1000, HID), _f32),
            pltpu.VMEM((LANE, 6), _f32),
            pltpu.VMEM((36, HID), _f32),
            pltpu.VMEM((2, LANE), _f32),
            pltpu.VMEM((LANE,), _f32),
            pltpu.VMEM((LANE, OUT), _f32),
            pltpu.VMEM((LANE, OUT), _f32),
        ],
    )(x, ta, tb, W1, degp)


def _edge_pipeline(g_hbm, srcf_hbm, dst_hbm, acc_sp,
                   sidx_v, didx_v, rows_v, gsems, ssems, base, nchunks):
    def FG(g, h):
        pltpu.async_copy(g_hbm.at[sidx_v.at[pl.ds(g * G * LANE, G * LANE)]],
                         rows_v.at[h], gsems[h])

    def WG(g, h):
        pltpu.make_async_copy(
            g_hbm.at[sidx_v.at[pl.ds(g * G * LANE, G * LANE)]],
            rows_v.at[h], gsems[h]).wait()

    def FS(g, h):
        for b in range(G):
            pltpu.async_copy(rows_v.at[h, pl.ds(b * LANE, LANE)],
                             acc_sp.at[didx_v.at[g * G + b]], ssems[h],
                             add=True)

    def WS(g, h):
        pltpu.make_async_copy(rows_v.at[h],
                              acc_sp.at[pl.ds(0, G * LANE)],
                              ssems[h]).wait()

    @pl.loop(0, nchunks)
    def _chunk(k):
        row0 = base + k * CH
        pltpu.sync_copy(srcf_hbm.at[pl.ds(row0 * LANE, CH * LANE)], sidx_v)
        pltpu.sync_copy(dst_hbm.at[pl.ds(row0, CH)], didx_v)
        FG(0, 0)
        FG(1, 1)

        @pl.loop(0, NGRP // 2 - 1)
        def _pair(p):
            g0 = 2 * p
            WG(g0, 0)
            FS(g0, 0)
            WG(g0 + 1, 1)
            FS(g0 + 1, 1)
            WS(g0, 0)
            FG(g0 + 2, 0)
            WS(g0 + 1, 1)
            FG(g0 + 3, 1)

        WG(NGRP - 2, 0)
        FS(NGRP - 2, 0)
        WG(NGRP - 1, 1)
        FS(NGRP - 1, 1)
        WS(NGRP - 2, 0)
        WS(NGRP - 1, 1)


def _sc_scatter1_body(src_hbm, dst_hbm, ga_hbm, gb_hbm, z_hbm, out_hbm,
                      sidx_v, didx_v, rows_v, gsem0, gsem1, ssem0, ssem1,
                      acc_sp):
    c = lax.axis_index("c")
    s = lax.axis_index("s")

    pltpu.sync_copy(z_hbm, acc_sp.at[pl.ds(s * NPT, NPT)])
    plsc.subcore_barrier()

    args = (src_hbm, dst_hbm, acc_sp, sidx_v, didx_v, rows_v,
            (gsem0, gsem1), (ssem0, ssem1))

    @pl.when(c == 0)
    def _half_a():
        _edge_pipeline(ga_hbm, *args,
                       base=s * TILE_E_ROWS2, nchunks=TILE_E_ROWS2 // CH)

    @pl.when(c == 1)
    def _half_b():
        _edge_pipeline(gb_hbm, *args,
                       base=s * TILE_E_ROWS2, nchunks=TILE_E_ROWS2 // CH)

    plsc.subcore_barrier()
    pltpu.sync_copy(acc_sp.at[pl.ds(s * NPT, NPT)],
                    out_hbm.at[c, pl.ds(s * NPT, NPT)])


def _sc_scatter1(srcf, dst_r, ga, gb):
    z = jnp.zeros((NPT, OUT), _f32)
    return pl.kernel(
        _sc_scatter1_body,
        out_type=jax.ShapeDtypeStruct((NC, N_PAD, OUT), _f32),
        mesh=_mesh,
        compiler_params=_sc_params,
        scratch_types=[
            pltpu.VMEM((CH * LANE,), _i32),
            pltpu.VMEM((CH, LANE), _i32),
            pltpu.VMEM((2, G * LANE, OUT), _f32),
            pltpu.SemaphoreType.DMA,
            pltpu.SemaphoreType.DMA,
            pltpu.SemaphoreType.DMA,
            pltpu.SemaphoreType.DMA,
            pltpu.VMEM_SHARED((N_PAD, OUT), _f32),
        ],
    )(srcf, dst_r, ga, gb, z)


def _sc_scatter2_body(src_hbm, dst_hbm, g_hbm, z_hbm, out_hbm,
                      sidx_v, didx_v, rows_v, gsem0, gsem1, ssem0, ssem1,
                      acc_sp):
    c = lax.axis_index("c")
    s = lax.axis_index("s")

    pltpu.sync_copy(z_hbm, acc_sp.at[pl.ds(s * NPT, NPT)])
    plsc.subcore_barrier()

    _edge_pipeline(g_hbm, src_hbm, dst_hbm, acc_sp, sidx_v, didx_v, rows_v,
                   (gsem0, gsem1), (ssem0, ssem1),
                   base=(c * NS + s) * TILE_E_ROWS,
                   nchunks=TILE_E_ROWS // CH)

    plsc.subcore_barrier()
    pltpu.sync_copy(acc_sp.at[pl.ds(s * NPT, NPT)],
                    out_hbm.at[c, pl.ds(s * NPT, NPT)])


def _sc_scatter2(srcf, dst_r, g):
    z = jnp.zeros((NPT, OUT), _f32)
    return pl.kernel(
        _sc_scatter2_body,
        out_type=jax.ShapeDtypeStruct((NC, N_PAD, OUT), _f32),
        mesh=_mesh,
        compiler_params=_sc_params,
        scratch_types=[
            pltpu.VMEM((CH * LANE,), _i32),
            pltpu.VMEM((CH, LANE), _i32),
            pltpu.VMEM((2, G * LANE, OUT), _f32),
            pltpu.SemaphoreType.DMA,
            pltpu.SemaphoreType.DMA,
            pltpu.SemaphoreType.DMA,
            pltpu.SemaphoreType.DMA,
            pltpu.VMEM_SHARED((N_PAD, OUT), _f32),
        ],
    )(srcf, dst_r, g, z)


def _sc_node2_body(s1_hbm, g1a_hbm, g1b_hbm, dis_hbm, b1_hbm, W2_hbm,
                   g2_hbm,
                   sa_v, sb_v, ga_v, gb_v, dis_v, o1_v, w2_v, b1_v, g2_v):
    c = lax.axis_index("c")
    s = lax.axis_index("s")
    w = s * NC + c

    pltpu.sync_copy(W2_hbm, w2_v)
    pltpu.sync_copy(b1_hbm, b1_v)
    b1a = b1_v[pl.ds(0, 16)]
    b1b = b1_v[pl.ds(16, 16)]
    zero = jnp.zeros((16,), _f32)
    iota = lax.iota(_i32, 16)

    @pl.loop(w * NV // NW, (w + 1) * NV // NW)
    def _row(r):
        node0 = jnp.minimum(r * LANE, N - LANE)
        sl = pl.ds(node0, LANE)
        pltpu.sync_copy(s1_hbm.at[0, sl], sa_v)
        pltpu.sync_copy(s1_hbm.at[1, sl], sb_v)
        pltpu.sync_copy(g1a_hbm.at[sl], ga_v)
        pltpu.sync_copy(g1b_hbm.at[sl], gb_v)
        pltpu.sync_copy(dis_hbm.at[sl], dis_v)

        @pl.loop(0, LANE // 16)
        def _grp(t):
            base16 = t * 16
            dis16 = dis_v[pl.ds(base16, 16)]
            for j in range(16):
                n = base16 + j
                dn = dis16[j]
                o1a = jnp.maximum((sa_v[n, :] + ga_v[n, :]) * dn + b1a, zero)
                o1b = jnp.maximum((sb_v[n, :] + gb_v[n, :]) * dn + b1b, zero)
                acc = zero
                for k in range(OUT):
                    acc = acc + o1a[k] * w2_v[k, :]
                for k in range(OUT):
                    acc = acc + o1b[k] * w2_v[OUT + k, :]
                g2_v[n, :] = acc * dn

        pltpu.sync_copy(g2_v, g2_hbm.at[sl])


def _sc_node2(s1, g1a, g1b, dis, b1, W2):
    return pl.kernel(
        _sc_node2_body,
        out_type=jax.ShapeDtypeStruct((N_PAD, OUT), _f32),
        mesh=_mesh,
        compiler_params=_sc_params,
        scratch_types=[
            pltpu.VMEM((LANE, OUT), _f32),
            pltpu.VMEM((LANE, OUT), _f32),
            pltpu.VMEM((LANE, OUT), _f32),
            pltpu.VMEM((LANE, OUT), _f32),
            pltpu.VMEM((LANE,), _f32),
            pltpu.VMEM((HID, 16), _f32),
            pltpu.VMEM((HID, OUT), _f32),
            pltpu.VMEM((HID,), _f32),
            pltpu.VMEM((LANE, OUT), _f32),
        ],
    )(s1, g1a, g1b, dis, b1, W2)


def _sc_node3_body(s2_hbm, g2_hbm, dis_hbm, b2_hbm, out_hbm,
                   sa_v, sb_v, g2_v, dis_v, b2_v, o_v):
    c = lax.axis_index("c")
    s = lax.axis_index("s")
    w = s * NC + c

    pltpu.sync_copy(b2_hbm, b2_v)
    b2r = b2_v[...]
    zero = jnp.zeros((16,), _f32)
    iota = lax.iota(_i32, 16)

    @pl.loop(w * NV // NW, (w + 1) * NV // NW)
    def _row(r):
        node0 = jnp.minimum(r * LANE, N - LANE)
        sl = pl.ds(node0, LANE)
        pltpu.sync_copy(s2_hbm.at[0, sl], sa_v)
        pltpu.sync_copy(s2_hbm.at[1, sl], sb_v)
        pltpu.sync_copy(g2_hbm.at[sl], g2_v)
        pltpu.sync_copy(dis_hbm.at[sl], dis_v)

        @pl.loop(0, LANE // 16)
        def _grp(t):
            base16 = t * 16
            dis16 = dis_v[pl.ds(base16, 16)]
            for j in range(16):
                n = base16 + j
                o_v[n, :] = jnp.maximum(
                    (sa_v[n, :] + sb_v[n, :] + g2_v[n, :]) * dis16[j] + b2r,
                    zero)

        pltpu.sync_copy(o_v, out_hbm.at[sl])


def _sc_node3(s2, g2, dis, b2):
    return pl.kernel(
        _sc_node3_body,
        out_type=jax.ShapeDtypeStruct((N, OUT), _f32),
        mesh=_mesh,
        compiler_params=_sc_params,
        scratch_types=[
            pltpu.VMEM((LANE, OUT), _f32),
            pltpu.VMEM((LANE, OUT), _f32),
            pltpu.VMEM((LANE, OUT), _f32),
            pltpu.VMEM((LANE,), _f32),
            pltpu.VMEM((OUT,), _f32),
            pltpu.VMEM((LANE, OUT), _f32),
        ],
    )(s2, g2, dis, b2)


def kernel(x, edge_index, emb_a, emb_b, W1, b1, W2, b2):
    src = edge_index[0].astype(_i32)
    dst = edge_index[1].astype(_i32)
    pad_e = E_ROWS * LANE - E
    pad_src = jnp.zeros((pad_e,), _i32)
    pad_dst = N + (jnp.arange(pad_e, dtype=_i32) % (N_PAD - N))
    srcf = jnp.concatenate([src, pad_src])
    dst_r = jnp.concatenate([dst, pad_dst]).reshape(E_ROWS, LANE)

    ta, tb = _tc_tables(emb_a, emb_b, W1)
    degp = _sc_deg(dst_r).reshape(NC, N_PAD)
    dis, g1a, g1b = _sc_node1(x, ta, tb, W1, degp)
    s1 = _sc_scatter1(srcf, dst_r, g1a, g1b)
    g2 = _sc_node2(s1, g1a, g1b, dis, b1, W2)
    s2 = _sc_scatter2(srcf, dst_r, g2)
    return _sc_node3(s2, g2, dis, b2)


# speedup vs baseline: 1.0453x; 1.0453x over previous
import functools

import jax
import jax.numpy as jnp
from jax import lax
from jax.experimental import pallas as pl
from jax.experimental.pallas import tpu as pltpu
from jax.experimental.pallas import tpu_sc as plsc

N = 50000
E = 1600000
OUT = 16
HID = 32
LANE = 128

NV = 391
N_ROWS = 400
N_PAD = N_ROWS * LANE
E_ROWS = 12544
NC, NS = 2, 16
NW = NC * NS
TILE_E_ROWS = E_ROWS // NW
TILE_E_ROWS2 = E_ROWS // NS
NPT = N_PAD // NS
CH = 56
G = 7
NGRP = CH // G

_mesh = plsc.VectorSubcoreMesh(core_axis_name="c", subcore_axis_name="s")
_f32 = jnp.float32
_i32 = jnp.int32
_sc_params = pltpu.CompilerParams(use_tc_tiling_on_sc=False,
                                  needs_layout_passes=False)


def _rsqrt16(d):
    xi = plsc.bitcast(d, _i32)
    y = plsc.bitcast(jnp.int32(0x5F3759DF) - (xi >> 1), _f32)
    for _ in range(3):
        y = y * (1.5 - 0.5 * d * y * y)
    return y


def _tc_tables_body(ea, eb, W1, ta_o, tb_o):
    ta_o[...] = jnp.dot(ea[...], W1[0:16, :], preferred_element_type=_f32)
    tb_o[...] = jnp.dot(eb[...], W1[16:32, :], preferred_element_type=_f32)


def _tc_tables(emb_a, emb_b, W1):
    return pl.pallas_call(
        _tc_tables_body,
        out_shape=[
            jax.ShapeDtypeStruct((1000, HID), _f32),
            jax.ShapeDtypeStruct((1000, HID), _f32),
        ],
    )(emb_a, emb_b, W1)


RPT = N_ROWS // 16


def _sc_deg_body(dst_hbm, z2_hbm, deg_hbm, idx_v, ridx_v, ld_v, deg_sp):
    c = lax.axis_index("c")
    s = lax.axis_index("s")
    ones16 = jnp.ones((16,), _f32)
    zeros16 = jnp.zeros((16,), _f32)
    iota = lax.iota(_i32, 16)

    @pl.loop(0, N_ROWS)
    def _z(r):
        for u in range(LANE // 16):
            ld_v[r, pl.ds(u * 16, 16)] = zeros16

    for k in range(RPT):
        ridx_v[k, :] = iota + k * 16

    pltpu.sync_copy(z2_hbm, deg_sp.at[pl.ds(s * RPT, RPT)])
    plsc.subcore_barrier()

    base = (c * NS + s) * TILE_E_ROWS

    @pl.loop(0, TILE_E_ROWS // CH)
    def _deg_chunk(k):
        pltpu.sync_copy(dst_hbm.at[pl.ds(base + k * CH, CH)], idx_v)

        @pl.loop(0, CH)
        def _deg_row(j):
            for u in range(LANE // 16):
                d16 = idx_v[j, pl.ds(u * 16, 16)]
                plsc.addupdate_scatter(ld_v, [d16 >> 7, d16 & 127], ones16)

    @pl.loop(0, RPT)
    def _merge(k):
        pltpu.sync_copy(ld_v.at[pl.ds(k * 16, 16)],
                        deg_sp.at[ridx_v.at[k]], add=True)

    plsc.subcore_barrier()
    pltpu.sync_copy(deg_sp.at[pl.ds(s * RPT, RPT)],
                    deg_hbm.at[c, pl.ds(s * RPT, RPT)])


def _sc_deg(dst_r):
    z2 = jnp.zeros((RPT, LANE), _f32)
    return pl.kernel(
        _sc_deg_body,
        out_type=jax.ShapeDtypeStruct((NC, N_ROWS, LANE), _f32),
        mesh=_mesh,
        compiler_params=_sc_params,
        scratch_types=[
            pltpu.VMEM((CH, LANE), _i32),
            pltpu.VMEM((RPT, 16), _i32),
            pltpu.VMEM((N_ROWS, LANE), _f32),
            pltpu.VMEM_SHARED((N_ROWS, LANE), _f32),
        ],
    )(dst_r, z2)


def _sc_node1_body(x_hbm, ta_hbm, tb_hbm, W1_hbm, degp_hbm,
                   dis_hbm, g1a_hbm, g1b_hbm,
                   ta_v, tb_v, x_v, w1_v, deg_v, dis_v, ga_v, gb_v):
    c = lax.axis_index("c")
    s = lax.axis_index("s")
    w = s * NC + c

    pltpu.sync_copy(ta_hbm, ta_v)
    pltpu.sync_copy(tb_hbm, tb_v)
    pltpu.sync_copy(W1_hbm, w1_v)
    w1a = [w1_v[32 + k, pl.ds(0, 16)] for k in range(4)]
    w1b = [w1_v[32 + k, pl.ds(16, 16)] for k in range(4)]
    iota = lax.iota(_i32, 16)

    @pl.loop(w * NV // NW, (w + 1) * NV // NW)
    def _row(r):
        node0 = jnp.minimum(r * LANE, N - LANE)
        pltpu.sync_copy(x_hbm.at[pl.ds(node0, LANE)], x_v)
        pltpu.sync_copy(degp_hbm.at[0, pl.ds(node0, LANE)], deg_v.at[0])
        pltpu.sync_copy(degp_hbm.at[1, pl.ds(node0, LANE)], deg_v.at[1])
        for v in range(LANE // 16):
            sl = pl.ds(v * 16, 16)
            d = deg_v[0, sl] + deg_v[1, sl] + 1.0
            dis_v[sl] = _rsqrt16(d)
        pltpu.sync_copy(dis_v, dis_hbm.at[pl.ds(node0, LANE)])

        @pl.loop(0, LANE // 16)
        def _grp(t):
            base16 = t * 16
            nidx = iota + base16
            dis16 = dis_v[pl.ds(base16, 16)]
            ia16 = plsc.load_gather(
                x_v, [nidx, jnp.zeros((16,), _i32)]).astype(_i32)
            ib16 = plsc.load_gather(
                x_v, [nidx, jnp.full((16,), 1, _i32)]).astype(_i32)
            nums = [plsc.load_gather(x_v, [nidx, jnp.full((16,), 2 + k, _i32)])
                    for k in range(4)]
            for j in range(16):
                ian = ia16[j]
                ibn = ib16[j]
                ha = ta_v[ian, pl.ds(0, 16)] + tb_v[ibn, pl.ds(0, 16)]
                hb = ta_v[ian, pl.ds(16, 16)] + tb_v[ibn, pl.ds(16, 16)]
                for k in range(4):
                    nk = nums[k][j]
                    ha = ha + nk * w1a[k]
                    hb = hb + nk * w1b[k]
                dn = dis16[j]
                ga_v[base16 + j, :] = ha * dn
                gb_v[base16 + j, :] = hb * dn

        pltpu.sync_copy(ga_v, g1a_hbm.at[pl.ds(node0, LANE)])
        pltpu.sync_copy(gb_v, g1b_hbm.at[pl.ds(node0, LANE)])


def _sc_node1(x, ta, tb, W1, degp):
    return pl.kernel(
        _sc_node1_body,
        out_type=[
            jax.ShapeDtypeStruct((N_PAD,), _f32),
            jax.ShapeDtypeStruct((N_PAD, OUT), _f32),
            jax.ShapeDtypeStruct((N_PAD, OUT), _f32),
        ],
        mesh=_mesh,
        compiler_params=_sc_params,
        scratch_types=[
            pltpu.VMEM((1000, HID), _f32),
            pltpu.VMEM((1000, HID), _f32),
            pltpu.VMEM((LANE, 6), _f32),
            pltpu.VMEM((36, HID), _f32),
            pltpu.VMEM((2, LANE), _f32),
            pltpu.VMEM((LANE,), _f32),
            pltpu.VMEM((LANE, OUT), _f32),
            pltpu.VMEM((LANE, OUT), _f32),
        ],
    )(x, ta, tb, W1, degp)


def _edge_pipeline(g_hbm, srcf_hbm, dst_hbm, acc_sp,
                   sidx_v, didx_v, rows_v, gsems, ssems, base, nchunks):
    def FG(g, h):
        pltpu.async_copy(g_hbm.at[sidx_v.at[pl.ds(g * G * LANE, G * LANE)]],
                         rows_v.at[h], gsems[h])

    def WG(g, h):
        pltpu.make_async_copy(
            g_hbm.at[sidx_v.at[pl.ds(g * G * LANE, G * LANE)]],
            rows_v.at[h], gsems[h]).wait()

    def FS(g, h):
        for b in range(G):
            pltpu.async_copy(rows_v.at[h, pl.ds(b * LANE, LANE)],
                             acc_sp.at[didx_v.at[g * G + b]], ssems[h],
                             add=True)

    def WS(g, h):
        pltpu.make_async_copy(rows_v.at[h],
                              acc_sp.at[pl.ds(0, G * LANE)],
                              ssems[h]).wait()

    @pl.loop(0, nchunks)
    def _chunk(k):
        row0 = base + k * CH
        pltpu.sync_copy(srcf_hbm.at[pl.ds(row0 * LANE, CH * LANE)], sidx_v)
        pltpu.sync_copy(dst_hbm.at[pl.ds(row0, CH)], didx_v)
        FG(0, 0)
        FG(1, 1)

        @pl.loop(0, NGRP // 2 - 1)
        def _pair(p):
            g0 = 2 * p
            WG(g0, 0)
            FS(g0, 0)
            WG(g0 + 1, 1)
            FS(g0 + 1, 1)
            WS(g0, 0)
            FG(g0 + 2, 0)
            WS(g0 + 1, 1)
            FG(g0 + 3, 1)

        WG(NGRP - 2, 0)
        FS(NGRP - 2, 0)
        WG(NGRP - 1, 1)
        FS(NGRP - 1, 1)
        WS(NGRP - 2, 0)
        WS(NGRP - 1, 1)


def _sc_scatter1_body(src_hbm, dst_hbm, ga_hbm, gb_hbm, z_hbm, out_hbm,
                      sidx_v, didx_v, rows_v, gsem0, gsem1, ssem0, ssem1,
                      acc_sp):
    c = lax.axis_index("c")
    s = lax.axis_index("s")

    pltpu.sync_copy(z_hbm, acc_sp.at[pl.ds(s * NPT, NPT)])
    plsc.subcore_barrier()

    args = (src_hbm, dst_hbm, acc_sp, sidx_v, didx_v, rows_v,
            (gsem0, gsem1), (ssem0, ssem1))

    @pl.when(c == 0)
    def _half_a():
        _edge_pipeline(ga_hbm, *args,
                       base=s * TILE_E_ROWS2, nchunks=TILE_E_ROWS2 // CH)

    @pl.when(c == 1)
    def _half_b():
        _edge_pipeline(gb_hbm, *args,
                       base=s * TILE_E_ROWS2, nchunks=TILE_E_ROWS2 // CH)

    plsc.subcore_barrier()
    pltpu.sync_copy(acc_sp.at[pl.ds(s * NPT, NPT)],
                    out_hbm.at[c, pl.ds(s * NPT, NPT)])


def _sc_scatter1(srcf, dst_r, ga, gb):
    z = jnp.zeros((NPT, OUT), _f32)
    return pl.kernel(
        _sc_scatter1_body,
        out_type=jax.ShapeDtypeStruct((NC, N_PAD, OUT), _f32),
        mesh=_mesh,
        compiler_params=_sc_params,
        scratch_types=[
            pltpu.VMEM((CH * LANE,), _i32),
            pltpu.VMEM((CH, LANE), _i32),
            pltpu.VMEM((2, G * LANE, OUT), _f32),
            pltpu.SemaphoreType.DMA,
            pltpu.SemaphoreType.DMA,
            pltpu.SemaphoreType.DMA,
            pltpu.SemaphoreType.DMA,
            pltpu.VMEM_SHARED((N_PAD, OUT), _f32),
        ],
    )(srcf, dst_r, ga, gb, z)


def _sc_scatter2_body(src_hbm, dst_hbm, g_hbm, z_hbm, out_hbm,
                      sidx_v, didx_v, rows_v, gsem0, gsem1, ssem0, ssem1,
                      acc_sp):
    c = lax.axis_index("c")
    s = lax.axis_index("s")

    pltpu.sync_copy(z_hbm, acc_sp.at[pl.ds(s * NPT, NPT)])
    plsc.subcore_barrier()

    _edge_pipeline(g_hbm, src_hbm, dst_hbm, acc_sp, sidx_v, didx_v, rows_v,
                   (gsem0, gsem1), (ssem0, ssem1),
                   base=(c * NS + s) * TILE_E_ROWS,
                   nchunks=TILE_E_ROWS // CH)

    plsc.subcore_barrier()
    pltpu.sync_copy(acc_sp.at[pl.ds(s * NPT, NPT)],
                    out_hbm.at[c, pl.ds(s * NPT, NPT)])


def _sc_scatter2(srcf, dst_r, g):
    z = jnp.zeros((NPT, OUT), _f32)
    return pl.kernel(
        _sc_scatter2_body,
        out_type=jax.ShapeDtypeStruct((NC, N_PAD, OUT), _f32),
        mesh=_mesh,
        compiler_params=_sc_params,
        scratch_types=[
            pltpu.VMEM((CH * LANE,), _i32),
            pltpu.VMEM((CH, LANE), _i32),
            pltpu.VMEM((2, G * LANE, OUT), _f32),
            pltpu.SemaphoreType.DMA,
            pltpu.SemaphoreType.DMA,
            pltpu.SemaphoreType.DMA,
            pltpu.SemaphoreType.DMA,
            pltpu.VMEM_SHARED((N_PAD, OUT), _f32),
        ],
    )(srcf, dst_r, g, z)


def _sc_node2_body(s1_hbm, g1a_hbm, g1b_hbm, dis_hbm, b1_hbm, W2_hbm,
                   g2_hbm,
                   sa_v, sb_v, ga_v, gb_v, dis_v, o1_v, w2_v, b1_v, g2_v):
    c = lax.axis_index("c")
    s = lax.axis_index("s")
    w = s * NC + c

    pltpu.sync_copy(W2_hbm, w2_v)
    pltpu.sync_copy(b1_hbm, b1_v)
    b1a = b1_v[pl.ds(0, 16)]
    b1b = b1_v[pl.ds(16, 16)]
    zero = jnp.zeros((16,), _f32)
    iota = lax.iota(_i32, 16)

    @pl.loop(w * NV // NW, (w + 1) * NV // NW)
    def _row(r):
        node0 = jnp.minimum(r * LANE, N - LANE)
        sl = pl.ds(node0, LANE)
        pltpu.sync_copy(s1_hbm.at[0, sl], sa_v)
        pltpu.sync_copy(s1_hbm.at[1, sl], sb_v)
        pltpu.sync_copy(g1a_hbm.at[sl], ga_v)
        pltpu.sync_copy(g1b_hbm.at[sl], gb_v)
        pltpu.sync_copy(dis_hbm.at[sl], dis_v)

        @pl.loop(0, LANE // 16)
        def _grp(t):
            base16 = t * 16
            dis16 = dis_v[pl.ds(base16, 16)]
            for j in range(16):
                n = base16 + j
                dn = dis16[j]
                o1a = jnp.maximum((sa_v[n, :] + ga_v[n, :]) * dn + b1a, zero)
                o1b = jnp.maximum((sb_v[n, :] + gb_v[n, :]) * dn + b1b, zero)
                acc = zero
                for k in range(OUT):
                    acc = acc + o1a[k] * w2_v[k, :]
                for k in range(OUT):
                    acc = acc + o1b[k] * w2_v[OUT + k, :]
                g2_v[n, :] = acc * dn

        pltpu.sync_copy(g2_v, g2_hbm.at[sl])


def _sc_node2(s1, g1a, g1b, dis, b1, W2):
    return pl.kernel(
        _sc_node2_body,
        out_type=jax.ShapeDtypeStruct((N_PAD, OUT), _f32),
        mesh=_mesh,
        compiler_params=_sc_params,
        scratch_types=[
            pltpu.VMEM((LANE, OUT), _f32),
            pltpu.VMEM((LANE, OUT), _f32),
            pltpu.VMEM((LANE, OUT), _f32),
            pltpu.VMEM((LANE, OUT), _f32),
            pltpu.VMEM((LANE,), _f32),
            pltpu.VMEM((HID, 16), _f32),
            pltpu.VMEM((HID, OUT), _f32),
            pltpu.VMEM((HID,), _f32),
            pltpu.VMEM((LANE, OUT), _f32),
        ],
    )(s1, g1a, g1b, dis, b1, W2)


def _sc_node3_body(s2_hbm, g2_hbm, dis_hbm, b2_hbm, out_hbm,
                   sa_v, sb_v, g2_v, dis_v, b2_v, o_v):
    c = lax.axis_index("c")
    s = lax.axis_index("s")
    w = s * NC + c

    pltpu.sync_copy(b2_hbm, b2_v)
    b2r = b2_v[...]
    zero = jnp.zeros((16,), _f32)
    iota = lax.iota(_i32, 16)

    @pl.loop(w * NV // NW, (w + 1) * NV // NW)
    def _row(r):
        node0 = jnp.minimum(r * LANE, N - LANE)
        sl = pl.ds(node0, LANE)
        pltpu.sync_copy(s2_hbm.at[0, sl], sa_v)
        pltpu.sync_copy(s2_hbm.at[1, sl], sb_v)
        pltpu.sync_copy(g2_hbm.at[sl], g2_v)
        pltpu.sync_copy(dis_hbm.at[sl], dis_v)

        @pl.loop(0, LANE // 16)
        def _grp(t):
            base16 = t * 16
            dis16 = dis_v[pl.ds(base16, 16)]
            for j in range(16):
                n = base16 + j
                o_v[n, :] = jnp.maximum(
                    (sa_v[n, :] + sb_v[n, :] + g2_v[n, :]) * dis16[j] + b2r,
                    zero)

        pltpu.sync_copy(o_v, out_hbm.at[sl])


def _sc_node3(s2, g2, dis, b2):
    return pl.kernel(
        _sc_node3_body,
        out_type=jax.ShapeDtypeStruct((N, OUT), _f32),
        mesh=_mesh,
        compiler_params=_sc_params,
        scratch_types=[
            pltpu.VMEM((LANE, OUT), _f32),
            pltpu.VMEM((LANE, OUT), _f32),
            pltpu.VMEM((LANE, OUT), _f32),
            pltpu.VMEM((LANE,), _f32),
            pltpu.VMEM((OUT,), _f32),
            pltpu.VMEM((LANE, OUT), _f32),
        ],
    )(s2, g2, dis, b2)


def kernel(x, edge_index, emb_a, emb_b, W1, b1, W2, b2):
    src = edge_index[0].astype(_i32)
    dst = edge_index[1].astype(_i32)
    pad_e = E_ROWS * LANE - E
    pad_src = jnp.zeros((pad_e,), _i32)
    pad_dst = N + (jnp.arange(pad_e, dtype=_i32) % (N_PAD - N))
    srcf = jnp.concatenate([src, pad_src])
    dst_r = jnp.concatenate([dst, pad_dst]).reshape(E_ROWS, LANE)

    ta, tb = _tc_tables(emb_a, emb_b, W1)
    degp = _sc_deg(dst_r).reshape(NC, N_PAD)
    dis, g1a, g1b = _sc_node1(x, ta, tb, W1, degp)
    s1 = _sc_scatter1(srcf, dst_r, g1a, g1b)
    g2 = _sc_node2(s1, g1a, g1b, dis, b1, W2)
    s2 = _sc_scatter2(srcf, dst_r, g2)
    return _sc_node3(s2, g2, dis, b2)


# speedup vs baseline: 1.1393x; 1.0899x over previous
import functools

import jax
import jax.numpy as jnp
from jax import lax
from jax.experimental import pallas as pl
from jax.experimental.pallas import tpu as pltpu
from jax.experimental.pallas import tpu_sc as plsc

N = 50000
E = 1600000
OUT = 16
HID = 32
LANE = 128

NV = 391
N_ROWS = 400
N_PAD = N_ROWS * LANE
E_ROWS = 12500
NC, NS = 2, 16
NW = NC * NS
NPT = N_PAD // NS
CH = 56
G = 7
NGRP = CH // G
NCHK = E_ROWS // CH
REM0 = NCHK * CH
REMR = E_ROWS - REM0

_mesh = plsc.VectorSubcoreMesh(core_axis_name="c", subcore_axis_name="s")
_f32 = jnp.float32
_i32 = jnp.int32
_sc_params = pltpu.CompilerParams(use_tc_tiling_on_sc=False,
                                  needs_layout_passes=False)


def _rsqrt16(d):
    xi = plsc.bitcast(d, _i32)
    y = plsc.bitcast(jnp.int32(0x5F3759DF) - (xi >> 1), _f32)
    for _ in range(3):
        y = y * (1.5 - 0.5 * d * y * y)
    return y


def _tc_tables_body(ea, eb, W1, ta_o, tb_o):
    ta_o[...] = jnp.dot(ea[...], W1[0:16, :], preferred_element_type=_f32)
    tb_o[...] = jnp.dot(eb[...], W1[16:32, :], preferred_element_type=_f32)


def _tc_tables(emb_a, emb_b, W1):
    return pl.pallas_call(
        _tc_tables_body,
        out_shape=[
            jax.ShapeDtypeStruct((1000, HID), _f32),
            jax.ShapeDtypeStruct((1000, HID), _f32),
        ],
    )(emb_a, emb_b, W1)


RPT = N_ROWS // 16


def _sc_deg_body(dst_hbm, z2_hbm, deg_hbm, idx_v, ridx_v, ld_v, deg_sp):
    c = lax.axis_index("c")
    s = lax.axis_index("s")
    ones16 = jnp.ones((16,), _f32)
    zeros16 = jnp.zeros((16,), _f32)
    iota = lax.iota(_i32, 16)

    @pl.loop(0, N_ROWS)
    def _z(r):
        for u in range(LANE // 16):
            ld_v[r, pl.ds(u * 16, 16)] = zeros16

    for k in range(RPT):
        ridx_v[k, :] = iota + k * 16

    pltpu.sync_copy(z2_hbm, deg_sp.at[pl.ds(s * RPT, RPT)])
    plsc.subcore_barrier()

    w = s * NC + c

    @pl.loop(w * NCHK // NW, (w + 1) * NCHK // NW)
    def _deg_chunk(k):
        pltpu.sync_copy(dst_hbm.at[pl.ds(k * CH, CH)], idx_v)

        @pl.loop(0, CH)
        def _deg_row(j):
            for u in range(LANE // 16):
                d16 = idx_v[j, pl.ds(u * 16, 16)]
                plsc.addupdate_scatter(ld_v, [d16 >> 7, d16 & 127], ones16)

    @pl.when(w == NW - 1)
    def _deg_rem():
        pltpu.sync_copy(dst_hbm.at[pl.ds(REM0, REMR)],
                        idx_v.at[pl.ds(0, REMR)])

        @pl.loop(0, REMR)
        def _deg_rem_row(j):
            for u in range(LANE // 16):
                d16 = idx_v[j, pl.ds(u * 16, 16)]
                plsc.addupdate_scatter(ld_v, [d16 >> 7, d16 & 127], ones16)

    @pl.loop(0, RPT)
    def _merge(k):
        pltpu.sync_copy(ld_v.at[pl.ds(k * 16, 16)],
                        deg_sp.at[ridx_v.at[k]], add=True)

    plsc.subcore_barrier()
    pltpu.sync_copy(deg_sp.at[pl.ds(s * RPT, RPT)],
                    deg_hbm.at[c, pl.ds(s * RPT, RPT)])


def _sc_deg(dst_r):
    z2 = jnp.zeros((RPT, LANE), _f32)
    return pl.kernel(
        _sc_deg_body,
        out_type=jax.ShapeDtypeStruct((NC, N_ROWS, LANE), _f32),
        mesh=_mesh,
        compiler_params=_sc_params,
        scratch_types=[
            pltpu.VMEM((CH, LANE), _i32),
            pltpu.VMEM((RPT, 16), _i32),
            pltpu.VMEM((N_ROWS, LANE), _f32),
            pltpu.VMEM_SHARED((N_ROWS, LANE), _f32),
        ],
    )(dst_r, z2)


def _sc_node1_body(x_hbm, ta_hbm, tb_hbm, W1_hbm, degp_hbm,
                   dis_hbm, g1a_hbm, g1b_hbm,
                   ta_v, tb_v, x_v, w1_v, deg_v, dis_v, ga_v, gb_v):
    c = lax.axis_index("c")
    s = lax.axis_index("s")
    w = s * NC + c

    pltpu.sync_copy(ta_hbm, ta_v)
    pltpu.sync_copy(tb_hbm, tb_v)
    pltpu.sync_copy(W1_hbm, w1_v)
    w1a = [w1_v[32 + k, pl.ds(0, 16)] for k in range(4)]
    w1b = [w1_v[32 + k, pl.ds(16, 16)] for k in range(4)]
    iota = lax.iota(_i32, 16)

    @pl.loop(w * NV // NW, (w + 1) * NV // NW)
    def _row(r):
        node0 = jnp.minimum(r * LANE, N - LANE)
        pltpu.sync_copy(x_hbm.at[pl.ds(node0, LANE)], x_v)
        pltpu.sync_copy(degp_hbm.at[0, pl.ds(node0, LANE)], deg_v.at[0])
        pltpu.sync_copy(degp_hbm.at[1, pl.ds(node0, LANE)], deg_v.at[1])
        for v in range(LANE // 16):
            sl = pl.ds(v * 16, 16)
            d = deg_v[0, sl] + deg_v[1, sl] + 1.0
            dis_v[sl] = _rsqrt16(d)
        pltpu.sync_copy(dis_v, dis_hbm.at[pl.ds(node0, LANE)])

        @pl.loop(0, LANE // 16)
        def _grp(t):
            base16 = t * 16
            nidx = iota + base16
            dis16 = dis_v[pl.ds(base16, 16)]
            ia16 = plsc.load_gather(
                x_v, [nidx, jnp.zeros((16,), _i32)]).astype(_i32)
            ib16 = plsc.load_gather(
                x_v, [nidx, jnp.full((16,), 1, _i32)]).astype(_i32)
            nums = [plsc.load_gather(x_v, [nidx, jnp.full((16,), 2 + k, _i32)])
                    for k in range(4)]
            for j in range(16):
                ian = ia16[j]
                ibn = ib16[j]
                ha = ta_v[ian, pl.ds(0, 16)] + tb_v[ibn, pl.ds(0, 16)]
                hb = ta_v[ian, pl.ds(16, 16)] + tb_v[ibn, pl.ds(16, 16)]
                for k in range(4):
                    nk = nums[k][j]
                    ha = ha + nk * w1a[k]
                    hb = hb + nk * w1b[k]
                dn = dis16[j]
                ga_v[base16 + j, :] = ha * dn
                gb_v[base16 + j, :] = hb * dn

        pltpu.sync_copy(ga_v, g1a_hbm.at[pl.ds(node0, LANE)])
        pltpu.sync_copy(gb_v, g1b_hbm.at[pl.ds(node0, LANE)])


def _sc_node1(x, ta, tb, W1, degp):
    return pl.kernel(
        _sc_node1_body,
        out_type=[
            jax.ShapeDtypeStruct((N_PAD,), _f32),
            jax.ShapeDtypeStruct((N_PAD, OUT), _f32),
            jax.ShapeDtypeStruct((N_PAD, OUT), _f32),
        ],
        mesh=_mesh,
        compiler_params=_sc_params,
        scratch_types=[
            pltpu.VMEM((1000, HID), _f32),
            pltpu.VMEM((1000, HID), _f32),
            pltpu.VMEM((LANE, 6), _f32),
            pltpu.VMEM((36, HID), _f32),
            pltpu.VMEM((2, LANE), _f32),
            pltpu.VMEM((LANE,), _f32),
            pltpu.VMEM((LANE, OUT), _f32),
            pltpu.VMEM((LANE, OUT), _f32),
        ],
    )(x, ta, tb, W1, degp)


def _edge_pipeline(g_hbm, srcf_hbm, dst_hbm, acc_sp,
                   sidx_v, didx_v, rows_v, gsems, ssems, chunk_lo, chunk_hi):
    def FG(g, h):
        pltpu.async_copy(g_hbm.at[sidx_v.at[pl.ds(g * G * LANE, G * LANE)]],
                         rows_v.at[h], gsems[h])

    def WG(g, h):
        pltpu.make_async_copy(
            g_hbm.at[sidx_v.at[pl.ds(g * G * LANE, G * LANE)]],
            rows_v.at[h], gsems[h]).wait()

    def FS(g, h):
        for b in range(G):
            pltpu.async_copy(rows_v.at[h, pl.ds(b * LANE, LANE)],
                             acc_sp.at[didx_v.at[g * G + b]], ssems[h],
                             add=True)

    def WS(g, h):
        pltpu.make_async_copy(rows_v.at[h],
                              acc_sp.at[pl.ds(0, G * LANE)],
                              ssems[h]).wait()

    @pl.loop(chunk_lo, chunk_hi)
    def _chunk(k):
        row0 = k * CH
        pltpu.sync_copy(srcf_hbm.at[pl.ds(row0 * LANE, CH * LANE)], sidx_v)
        pltpu.sync_copy(dst_hbm.at[pl.ds(row0, CH)], didx_v)
        FG(0, 0)
        FG(1, 1)

        @pl.loop(0, NGRP // 2 - 1)
        def _pair(p):
            g0 = 2 * p
            WG(g0, 0)
            FS(g0, 0)
            WG(g0 + 1, 1)
            FS(g0 + 1, 1)
            WS(g0, 0)
            FG(g0 + 2, 0)
            WS(g0 + 1, 1)
            FG(g0 + 3, 1)

        WG(NGRP - 2, 0)
        FS(NGRP - 2, 0)
        WG(NGRP - 1, 1)
        FS(NGRP - 1, 1)
        WS(NGRP - 2, 0)
        WS(NGRP - 1, 1)


def _edge_remainder(g_hbm, srcf_hbm, dst_hbm, acc_sp, sidx_v, didx_v,
                    rows_v, gsem):
    hw = REMR // 2
    pltpu.sync_copy(srcf_hbm.at[pl.ds(REM0 * LANE, REMR * LANE)],
                    sidx_v.at[pl.ds(0, REMR * LANE)])
    pltpu.sync_copy(dst_hbm.at[pl.ds(REM0, REMR)], didx_v.at[pl.ds(0, REMR)])
    for p in range(2):
        cp = pltpu.async_copy(
            g_hbm.at[sidx_v.at[pl.ds(p * hw * LANE, hw * LANE)]],
            rows_v.at[0, pl.ds(0, hw * LANE)], gsem)
        cp.wait()
        for b in range(hw):
            pltpu.sync_copy(rows_v.at[0, pl.ds(b * LANE, LANE)],
                            acc_sp.at[didx_v.at[p * hw + b]], add=True)


def _sc_scatter1_body(src_hbm, dst_hbm, ga_hbm, gb_hbm, z_hbm, out_hbm,
                      sidx_v, didx_v, rows_v, gsem0, gsem1, ssem0, ssem1,
                      acc_sp):
    c = lax.axis_index("c")
    s = lax.axis_index("s")

    pltpu.sync_copy(z_hbm, acc_sp.at[pl.ds(s * NPT, NPT)])
    plsc.subcore_barrier()

    args = (src_hbm, dst_hbm, acc_sp, sidx_v, didx_v, rows_v,
            (gsem0, gsem1), (ssem0, ssem1))

    lo = s * NCHK // NS
    hi = (s + 1) * NCHK // NS

    @pl.when(c == 0)
    def _half_a():
        _edge_pipeline(ga_hbm, *args, chunk_lo=lo, chunk_hi=hi)

        @pl.when(s == NS - 1)
        def _rem_a():
            _edge_remainder(ga_hbm, src_hbm, dst_hbm, acc_sp,
                            sidx_v, didx_v, rows_v, gsem0)

    @pl.when(c == 1)
    def _half_b():
        _edge_pipeline(gb_hbm, *args, chunk_lo=lo, chunk_hi=hi)

        @pl.when(s == NS - 1)
        def _rem_b():
            _edge_remainder(gb_hbm, src_hbm, dst_hbm, acc_sp,
                            sidx_v, didx_v, rows_v, gsem0)

    plsc.subcore_barrier()
    pltpu.sync_copy(acc_sp.at[pl.ds(s * NPT, NPT)],
                    out_hbm.at[c, pl.ds(s * NPT, NPT)])


def _sc_scatter1(srcf, dst_r, ga, gb):
    z = jnp.zeros((NPT, OUT), _f32)
    return pl.kernel(
        _sc_scatter1_body,
        out_type=jax.ShapeDtypeStruct((NC, N_PAD, OUT), _f32),
        mesh=_mesh,
        compiler_params=_sc_params,
        scratch_types=[
            pltpu.VMEM((CH * LANE,), _i32),
            pltpu.VMEM((CH, LANE), _i32),
            pltpu.VMEM((2, G * LANE, OUT), _f32),
            pltpu.SemaphoreType.DMA,
            pltpu.SemaphoreType.DMA,
            pltpu.SemaphoreType.DMA,
            pltpu.SemaphoreType.DMA,
            pltpu.VMEM_SHARED((N_PAD, OUT), _f32),
        ],
    )(srcf, dst_r, ga, gb, z)


def _sc_scatter2_body(src_hbm, dst_hbm, g_hbm, z_hbm, out_hbm,
                      sidx_v, didx_v, rows_v, gsem0, gsem1, ssem0, ssem1,
                      acc_sp):
    c = lax.axis_index("c")
    s = lax.axis_index("s")

    pltpu.sync_copy(z_hbm, acc_sp.at[pl.ds(s * NPT, NPT)])
    plsc.subcore_barrier()

    w = s * NC + c
    _edge_pipeline(g_hbm, src_hbm, dst_hbm, acc_sp, sidx_v, didx_v, rows_v,
                   (gsem0, gsem1), (ssem0, ssem1),
                   chunk_lo=w * NCHK // NW, chunk_hi=(w + 1) * NCHK // NW)

    @pl.when(w == NW - 1)
    def _rem():
        _edge_remainder(g_hbm, src_hbm, dst_hbm, acc_sp,
                        sidx_v, didx_v, rows_v, gsem0)

    plsc.subcore_barrier()
    pltpu.sync_copy(acc_sp.at[pl.ds(s * NPT, NPT)],
                    out_hbm.at[c, pl.ds(s * NPT, NPT)])


def _sc_scatter2(srcf, dst_r, g):
    z = jnp.zeros((NPT, OUT), _f32)
    return pl.kernel(
        _sc_scatter2_body,
        out_type=jax.ShapeDtypeStruct((NC, N_PAD, OUT), _f32),
        mesh=_mesh,
        compiler_params=_sc_params,
        scratch_types=[
            pltpu.VMEM((CH * LANE,), _i32),
            pltpu.VMEM((CH, LANE), _i32),
            pltpu.VMEM((2, G * LANE, OUT), _f32),
            pltpu.SemaphoreType.DMA,
            pltpu.SemaphoreType.DMA,
            pltpu.SemaphoreType.DMA,
            pltpu.SemaphoreType.DMA,
            pltpu.VMEM_SHARED((N_PAD, OUT), _f32),
        ],
    )(srcf, dst_r, g, z)


def _sc_node2_body(s1_hbm, g1a_hbm, g1b_hbm, dis_hbm, b1_hbm, W2_hbm,
                   g2_hbm,
                   sa_v, sb_v, ga_v, gb_v, dis_v, o1_v, w2_v, b1_v, g2_v):
    c = lax.axis_index("c")
    s = lax.axis_index("s")
    w = s * NC + c

    pltpu.sync_copy(W2_hbm, w2_v)
    pltpu.sync_copy(b1_hbm, b1_v)
    b1a = b1_v[pl.ds(0, 16)]
    b1b = b1_v[pl.ds(16, 16)]
    zero = jnp.zeros((16,), _f32)
    iota = lax.iota(_i32, 16)

    @pl.loop(w * NV // NW, (w + 1) * NV // NW)
    def _row(r):
        node0 = jnp.minimum(r * LANE, N - LANE)
        sl = pl.ds(node0, LANE)
        pltpu.sync_copy(s1_hbm.at[0, sl], sa_v)
        pltpu.sync_copy(s1_hbm.at[1, sl], sb_v)
        pltpu.sync_copy(g1a_hbm.at[sl], ga_v)
        pltpu.sync_copy(g1b_hbm.at[sl], gb_v)
        pltpu.sync_copy(dis_hbm.at[sl], dis_v)

        @pl.loop(0, LANE // 16)
        def _grp(t):
            base16 = t * 16
            dis16 = dis_v[pl.ds(base16, 16)]
            for j in range(16):
                n = base16 + j
                dn = dis16[j]
                o1a = jnp.maximum((sa_v[n, :] + ga_v[n, :]) * dn + b1a, zero)
                o1b = jnp.maximum((sb_v[n, :] + gb_v[n, :]) * dn + b1b, zero)
                acc = zero
                for k in range(OUT):
                    acc = acc + o1a[k] * w2_v[k, :]
                for k in range(OUT):
                    acc = acc + o1b[k] * w2_v[OUT + k, :]
                g2_v[n, :] = acc * dn

        pltpu.sync_copy(g2_v, g2_hbm.at[sl])


def _sc_node2(s1, g1a, g1b, dis, b1, W2):
    return pl.kernel(
        _sc_node2_body,
        out_type=jax.ShapeDtypeStruct((N_PAD, OUT), _f32),
        mesh=_mesh,
        compiler_params=_sc_params,
        scratch_types=[
            pltpu.VMEM((LANE, OUT), _f32),
            pltpu.VMEM((LANE, OUT), _f32),
            pltpu.VMEM((LANE, OUT), _f32),
            pltpu.VMEM((LANE, OUT), _f32),
            pltpu.VMEM((LANE,), _f32),
            pltpu.VMEM((HID, 16), _f32),
            pltpu.VMEM((HID, OUT), _f32),
            pltpu.VMEM((HID,), _f32),
            pltpu.VMEM((LANE, OUT), _f32),
        ],
    )(s1, g1a, g1b, dis, b1, W2)


def _sc_node3_body(s2_hbm, g2_hbm, dis_hbm, b2_hbm, out_hbm,
                   sa_v, sb_v, g2_v, dis_v, b2_v, o_v):
    c = lax.axis_index("c")
    s = lax.axis_index("s")
    w = s * NC + c

    pltpu.sync_copy(b2_hbm, b2_v)
    b2r = b2_v[...]
    zero = jnp.zeros((16,), _f32)
    iota = lax.iota(_i32, 16)

    @pl.loop(w * NV // NW, (w + 1) * NV // NW)
    def _row(r):
        node0 = jnp.minimum(r * LANE, N - LANE)
        sl = pl.ds(node0, LANE)
        pltpu.sync_copy(s2_hbm.at[0, sl], sa_v)
        pltpu.sync_copy(s2_hbm.at[1, sl], sb_v)
        pltpu.sync_copy(g2_hbm.at[sl], g2_v)
        pltpu.sync_copy(dis_hbm.at[sl], dis_v)

        @pl.loop(0, LANE // 16)
        def _grp(t):
            base16 = t * 16
            dis16 = dis_v[pl.ds(base16, 16)]
            for j in range(16):
                n = base16 + j
                o_v[n, :] = jnp.maximum(
                    (sa_v[n, :] + sb_v[n, :] + g2_v[n, :]) * dis16[j] + b2r,
                    zero)

        pltpu.sync_copy(o_v, out_hbm.at[sl])


def _sc_node3(s2, g2, dis, b2):
    return pl.kernel(
        _sc_node3_body,
        out_type=jax.ShapeDtypeStruct((N, OUT), _f32),
        mesh=_mesh,
        compiler_params=_sc_params,
        scratch_types=[
            pltpu.VMEM((LANE, OUT), _f32),
            pltpu.VMEM((LANE, OUT), _f32),
            pltpu.VMEM((LANE, OUT), _f32),
            pltpu.VMEM((LANE,), _f32),
            pltpu.VMEM((OUT,), _f32),
            pltpu.VMEM((LANE, OUT), _f32),
        ],
    )(s2, g2, dis, b2)


def kernel(x, edge_index, emb_a, emb_b, W1, b1, W2, b2):
    srcf = edge_index[0].astype(_i32)
    dst_r = edge_index[1].astype(_i32).reshape(E_ROWS, LANE)

    ta, tb = _tc_tables(emb_a, emb_b, W1)
    degp = _sc_deg(dst_r).reshape(NC, N_PAD)
    dis, g1a, g1b = _sc_node1(x, ta, tb, W1, degp)
    s1 = _sc_scatter1(srcf, dst_r, g1a, g1b)
    g2 = _sc_node2(s1, g1a, g1b, dis, b1, W2)
    s2 = _sc_scatter2(srcf, dst_r, g2)
    return _sc_node3(s2, g2, dis, b2)


# speedup vs baseline: 1.1959x; 1.0497x over previous
import functools

import jax
import jax.numpy as jnp
from jax import lax
from jax.experimental import pallas as pl
from jax.experimental.pallas import tpu as pltpu
from jax.experimental.pallas import tpu_sc as plsc

N = 50000
E = 1600000
OUT = 16
HID = 32
LANE = 128

NV = 391
BN = 512
NVB = 98
N_ROWS = 400
N_PAD = N_ROWS * LANE
E_ROWS = 12500
NC, NS = 2, 16
NW = NC * NS
NPT = N_PAD // NS
CH = 56
G = 7
NGRP = CH // G
NCHK = E_ROWS // CH
REM0 = NCHK * CH
REMR = E_ROWS - REM0

_mesh = plsc.VectorSubcoreMesh(core_axis_name="c", subcore_axis_name="s")
_f32 = jnp.float32
_i32 = jnp.int32
_sc_params = pltpu.CompilerParams(use_tc_tiling_on_sc=False,
                                  needs_layout_passes=False)


def _rsqrt16(d):
    xi = plsc.bitcast(d, _i32)
    y = plsc.bitcast(jnp.int32(0x5F3759DF) - (xi >> 1), _f32)
    for _ in range(3):
        y = y * (1.5 - 0.5 * d * y * y)
    return y


def _tc_tables_body(ea, eb, W1, ta_o, tb_o):
    ta_o[...] = jnp.dot(ea[...], W1[0:16, :], preferred_element_type=_f32)
    tb_o[...] = jnp.dot(eb[...], W1[16:32, :], preferred_element_type=_f32)


def _tc_tables(emb_a, emb_b, W1):
    return pl.pallas_call(
        _tc_tables_body,
        out_shape=[
            jax.ShapeDtypeStruct((1000, HID), _f32),
            jax.ShapeDtypeStruct((1000, HID), _f32),
        ],
    )(emb_a, emb_b, W1)


RPT = N_ROWS // 16


def _sc_deg_body(dst_hbm, z2_hbm, deg_hbm, idx_v, ridx_v, ld_v, deg_sp):
    c = lax.axis_index("c")
    s = lax.axis_index("s")
    ones16 = jnp.ones((16,), _f32)
    zeros16 = jnp.zeros((16,), _f32)
    iota = lax.iota(_i32, 16)

    @pl.loop(0, N_ROWS)
    def _z(r):
        for u in range(LANE // 16):
            ld_v[r, pl.ds(u * 16, 16)] = zeros16

    for k in range(RPT):
        ridx_v[k, :] = iota + k * 16

    pltpu.sync_copy(z2_hbm, deg_sp.at[pl.ds(s * RPT, RPT)])
    plsc.subcore_barrier()

    w = s * NC + c

    @pl.loop(w * NCHK // NW, (w + 1) * NCHK // NW)
    def _deg_chunk(k):
        pltpu.sync_copy(dst_hbm.at[pl.ds(k * CH, CH)], idx_v)

        @pl.loop(0, CH)
        def _deg_row(j):
            for u in range(LANE // 16):
                d16 = idx_v[j, pl.ds(u * 16, 16)]
                plsc.addupdate_scatter(ld_v, [d16 >> 7, d16 & 127], ones16)

    @pl.when(w == NW - 1)
    def _deg_rem():
        pltpu.sync_copy(dst_hbm.at[pl.ds(REM0, REMR)],
                        idx_v.at[pl.ds(0, REMR)])

        @pl.loop(0, REMR)
        def _deg_rem_row(j):
            for u in range(LANE // 16):
                d16 = idx_v[j, pl.ds(u * 16, 16)]
                plsc.addupdate_scatter(ld_v, [d16 >> 7, d16 & 127], ones16)

    @pl.loop(0, RPT)
    def _merge(k):
        pltpu.sync_copy(ld_v.at[pl.ds(k * 16, 16)],
                        deg_sp.at[ridx_v.at[k]], add=True)

    plsc.subcore_barrier()
    pltpu.sync_copy(deg_sp.at[pl.ds(s * RPT, RPT)],
                    deg_hbm.at[c, pl.ds(s * RPT, RPT)])


def _sc_deg(dst_r):
    z2 = jnp.zeros((RPT, LANE), _f32)
    return pl.kernel(
        _sc_deg_body,
        out_type=jax.ShapeDtypeStruct((NC, N_ROWS, LANE), _f32),
        mesh=_mesh,
        compiler_params=_sc_params,
        scratch_types=[
            pltpu.VMEM((CH, LANE), _i32),
            pltpu.VMEM((RPT, 16), _i32),
            pltpu.VMEM((N_ROWS, LANE), _f32),
            pltpu.VMEM_SHARED((N_ROWS, LANE), _f32),
        ],
    )(dst_r, z2)


def _sc_node1_body(x_hbm, ta_hbm, tb_hbm, W1_hbm, degp_hbm,
                   dis_hbm, g1a_hbm, g1b_hbm,
                   ta_v, tb_v, x_v, w1_v, deg_v, dis_v, ga_v, gb_v):
    c = lax.axis_index("c")
    s = lax.axis_index("s")
    w = s * NC + c

    pltpu.sync_copy(ta_hbm, ta_v)
    pltpu.sync_copy(tb_hbm, tb_v)
    pltpu.sync_copy(W1_hbm, w1_v)
    w1a = [w1_v[32 + k, pl.ds(0, 16)] for k in range(4)]
    w1b = [w1_v[32 + k, pl.ds(16, 16)] for k in range(4)]
    iota = lax.iota(_i32, 16)

    @pl.loop(w * NVB // NW, (w + 1) * NVB // NW)
    def _row(r):
        node0 = jnp.minimum(r * BN, N - BN)
        pltpu.sync_copy(x_hbm.at[pl.ds(node0, BN)], x_v)
        pltpu.sync_copy(degp_hbm.at[0, pl.ds(node0, BN)], deg_v.at[0])
        pltpu.sync_copy(degp_hbm.at[1, pl.ds(node0, BN)], deg_v.at[1])
        for v in range(BN // 16):
            sl = pl.ds(v * 16, 16)
            d = deg_v[0, sl] + deg_v[1, sl] + 1.0
            dis_v[sl] = _rsqrt16(d)
        pltpu.sync_copy(dis_v, dis_hbm.at[pl.ds(node0, BN)])

        @pl.loop(0, BN // 16)
        def _grp(t):
            base16 = t * 16
            nidx = iota + base16
            dis16 = dis_v[pl.ds(base16, 16)]
            ia16 = plsc.load_gather(
                x_v, [nidx, jnp.zeros((16,), _i32)]).astype(_i32)
            ib16 = plsc.load_gather(
                x_v, [nidx, jnp.full((16,), 1, _i32)]).astype(_i32)
            nums = [plsc.load_gather(x_v, [nidx, jnp.full((16,), 2 + k, _i32)])
                    for k in range(4)]
            for j in range(16):
                ian = ia16[j]
                ibn = ib16[j]
                ha = ta_v[ian, pl.ds(0, 16)] + tb_v[ibn, pl.ds(0, 16)]
                hb = ta_v[ian, pl.ds(16, 16)] + tb_v[ibn, pl.ds(16, 16)]
                for k in range(4):
                    nk = nums[k][j]
                    ha = ha + nk * w1a[k]
                    hb = hb + nk * w1b[k]
                dn = dis16[j]
                ga_v[base16 + j, :] = ha * dn
                gb_v[base16 + j, :] = hb * dn

        pltpu.sync_copy(ga_v, g1a_hbm.at[pl.ds(node0, BN)])
        pltpu.sync_copy(gb_v, g1b_hbm.at[pl.ds(node0, BN)])


def _sc_node1(x, ta, tb, W1, degp):
    return pl.kernel(
        _sc_node1_body,
        out_type=[
            jax.ShapeDtypeStruct((N_PAD,), _f32),
            jax.ShapeDtypeStruct((N_PAD, OUT), _f32),
            jax.ShapeDtypeStruct((N_PAD, OUT), _f32),
        ],
        mesh=_mesh,
        compiler_params=_sc_params,
        scratch_types=[
            pltpu.VMEM((1000, HID), _f32),
            pltpu.VMEM((1000, HID), _f32),
            pltpu.VMEM((BN, 6), _f32),
            pltpu.VMEM((36, HID), _f32),
            pltpu.VMEM((2, BN), _f32),
            pltpu.VMEM((BN,), _f32),
            pltpu.VMEM((BN, OUT), _f32),
            pltpu.VMEM((BN, OUT), _f32),
        ],
    )(x, ta, tb, W1, degp)


def _edge_pipeline(g_hbm, srcf_hbm, dst_hbm, acc_sp,
                   sidx_v, didx_v, rows_v, gsems, ssems, chunk_lo, chunk_hi):
    def FG(g, h):
        pltpu.async_copy(g_hbm.at[sidx_v.at[pl.ds(g * G * LANE, G * LANE)]],
                         rows_v.at[h], gsems[h])

    def WG(g, h):
        pltpu.make_async_copy(
            g_hbm.at[sidx_v.at[pl.ds(g * G * LANE, G * LANE)]],
            rows_v.at[h], gsems[h]).wait()

    def FS(g, h):
        for b in range(G):
            pltpu.async_copy(rows_v.at[h, pl.ds(b * LANE, LANE)],
                             acc_sp.at[didx_v.at[g * G + b]], ssems[h],
                             add=True)

    def WS(g, h):
        pltpu.make_async_copy(rows_v.at[h],
                              acc_sp.at[pl.ds(0, G * LANE)],
                              ssems[h]).wait()

    @pl.loop(chunk_lo, chunk_hi)
    def _chunk(k):
        row0 = k * CH
        pltpu.sync_copy(srcf_hbm.at[pl.ds(row0 * LANE, CH * LANE)], sidx_v)
        pltpu.sync_copy(dst_hbm.at[pl.ds(row0, CH)], didx_v)
        FG(0, 0)
        FG(1, 1)

        @pl.loop(0, NGRP // 2 - 1)
        def _pair(p):
            g0 = 2 * p
            WG(g0, 0)
            FS(g0, 0)
            WG(g0 + 1, 1)
            FS(g0 + 1, 1)
            WS(g0, 0)
            FG(g0 + 2, 0)
            WS(g0 + 1, 1)
            FG(g0 + 3, 1)

        WG(NGRP - 2, 0)
        FS(NGRP - 2, 0)
        WG(NGRP - 1, 1)
        FS(NGRP - 1, 1)
        WS(NGRP - 2, 0)
        WS(NGRP - 1, 1)


def _edge_remainder(g_hbm, srcf_hbm, dst_hbm, acc_sp, sidx_v, didx_v,
                    rows_v, gsem):
    hw = REMR // 2
    pltpu.sync_copy(srcf_hbm.at[pl.ds(REM0 * LANE, REMR * LANE)],
                    sidx_v.at[pl.ds(0, REMR * LANE)])
    pltpu.sync_copy(dst_hbm.at[pl.ds(REM0, REMR)], didx_v.at[pl.ds(0, REMR)])
    for p in range(2):
        cp = pltpu.async_copy(
            g_hbm.at[sidx_v.at[pl.ds(p * hw * LANE, hw * LANE)]],
            rows_v.at[0, pl.ds(0, hw * LANE)], gsem)
        cp.wait()
        for b in range(hw):
            pltpu.sync_copy(rows_v.at[0, pl.ds(b * LANE, LANE)],
                            acc_sp.at[didx_v.at[p * hw + b]], add=True)


def _sc_scatter1_body(src_hbm, dst_hbm, ga_hbm, gb_hbm, z_hbm, out_hbm,
                      sidx_v, didx_v, rows_v, gsem0, gsem1, ssem0, ssem1,
                      acc_sp):
    c = lax.axis_index("c")
    s = lax.axis_index("s")

    pltpu.sync_copy(z_hbm, acc_sp.at[pl.ds(s * NPT, NPT)])
    plsc.subcore_barrier()

    args = (src_hbm, dst_hbm, acc_sp, sidx_v, didx_v, rows_v,
            (gsem0, gsem1), (ssem0, ssem1))

    lo = s * NCHK // NS
    hi = (s + 1) * NCHK // NS

    @pl.when(c == 0)
    def _half_a():
        _edge_pipeline(ga_hbm, *args, chunk_lo=lo, chunk_hi=hi)

        @pl.when(s == NS - 1)
        def _rem_a():
            _edge_remainder(ga_hbm, src_hbm, dst_hbm, acc_sp,
                            sidx_v, didx_v, rows_v, gsem0)

    @pl.when(c == 1)
    def _half_b():
        _edge_pipeline(gb_hbm, *args, chunk_lo=lo, chunk_hi=hi)

        @pl.when(s == NS - 1)
        def _rem_b():
            _edge_remainder(gb_hbm, src_hbm, dst_hbm, acc_sp,
                            sidx_v, didx_v, rows_v, gsem0)

    plsc.subcore_barrier()
    pltpu.sync_copy(acc_sp.at[pl.ds(s * NPT, NPT)],
                    out_hbm.at[c, pl.ds(s * NPT, NPT)])


def _sc_scatter1(srcf, dst_r, ga, gb):
    z = jnp.zeros((NPT, OUT), _f32)
    return pl.kernel(
        _sc_scatter1_body,
        out_type=jax.ShapeDtypeStruct((NC, N_PAD, OUT), _f32),
        mesh=_mesh,
        compiler_params=_sc_params,
        scratch_types=[
            pltpu.VMEM((CH * LANE,), _i32),
            pltpu.VMEM((CH, LANE), _i32),
            pltpu.VMEM((2, G * LANE, OUT), _f32),
            pltpu.SemaphoreType.DMA,
            pltpu.SemaphoreType.DMA,
            pltpu.SemaphoreType.DMA,
            pltpu.SemaphoreType.DMA,
            pltpu.VMEM_SHARED((N_PAD, OUT), _f32),
        ],
    )(srcf, dst_r, ga, gb, z)


def _sc_scatter2_body(src_hbm, dst_hbm, g_hbm, z_hbm, out_hbm,
                      sidx_v, didx_v, rows_v, gsem0, gsem1, ssem0, ssem1,
                      acc_sp):
    c = lax.axis_index("c")
    s = lax.axis_index("s")

    pltpu.sync_copy(z_hbm, acc_sp.at[pl.ds(s * NPT, NPT)])
    plsc.subcore_barrier()

    w = s * NC + c
    _edge_pipeline(g_hbm, src_hbm, dst_hbm, acc_sp, sidx_v, didx_v, rows_v,
                   (gsem0, gsem1), (ssem0, ssem1),
                   chunk_lo=w * NCHK // NW, chunk_hi=(w + 1) * NCHK // NW)

    @pl.when(w == NW - 1)
    def _rem():
        _edge_remainder(g_hbm, src_hbm, dst_hbm, acc_sp,
                        sidx_v, didx_v, rows_v, gsem0)

    plsc.subcore_barrier()
    pltpu.sync_copy(acc_sp.at[pl.ds(s * NPT, NPT)],
                    out_hbm.at[c, pl.ds(s * NPT, NPT)])


def _sc_scatter2(srcf, dst_r, g):
    z = jnp.zeros((NPT, OUT), _f32)
    return pl.kernel(
        _sc_scatter2_body,
        out_type=jax.ShapeDtypeStruct((NC, N_PAD, OUT), _f32),
        mesh=_mesh,
        compiler_params=_sc_params,
        scratch_types=[
            pltpu.VMEM((CH * LANE,), _i32),
            pltpu.VMEM((CH, LANE), _i32),
            pltpu.VMEM((2, G * LANE, OUT), _f32),
            pltpu.SemaphoreType.DMA,
            pltpu.SemaphoreType.DMA,
            pltpu.SemaphoreType.DMA,
            pltpu.SemaphoreType.DMA,
            pltpu.VMEM_SHARED((N_PAD, OUT), _f32),
        ],
    )(srcf, dst_r, g, z)


def _sc_node2_body(s1_hbm, g1a_hbm, g1b_hbm, dis_hbm, b1_hbm, W2_hbm,
                   g2_hbm,
                   sa_v, sb_v, ga_v, gb_v, dis_v, o1_v, w2_v, b1_v, g2_v):
    c = lax.axis_index("c")
    s = lax.axis_index("s")
    w = s * NC + c

    pltpu.sync_copy(W2_hbm, w2_v)
    pltpu.sync_copy(b1_hbm, b1_v)
    b1a = b1_v[pl.ds(0, 16)]
    b1b = b1_v[pl.ds(16, 16)]
    zero = jnp.zeros((16,), _f32)
    iota = lax.iota(_i32, 16)

    @pl.loop(w * NVB // NW, (w + 1) * NVB // NW)
    def _row(r):
        node0 = jnp.minimum(r * BN, N - BN)
        sl = pl.ds(node0, BN)
        pltpu.sync_copy(s1_hbm.at[0, sl], sa_v)
        pltpu.sync_copy(s1_hbm.at[1, sl], sb_v)
        pltpu.sync_copy(g1a_hbm.at[sl], ga_v)
        pltpu.sync_copy(g1b_hbm.at[sl], gb_v)
        pltpu.sync_copy(dis_hbm.at[sl], dis_v)

        @pl.loop(0, BN // 16)
        def _grp(t):
            base16 = t * 16
            dis16 = dis_v[pl.ds(base16, 16)]
            for j in range(16):
                n = base16 + j
                dn = dis16[j]
                o1a = jnp.maximum((sa_v[n, :] + ga_v[n, :]) * dn + b1a, zero)
                o1b = jnp.maximum((sb_v[n, :] + gb_v[n, :]) * dn + b1b, zero)
                acc = zero
                for k in range(OUT):
                    acc = acc + o1a[k] * w2_v[k, :]
                for k in range(OUT):
                    acc = acc + o1b[k] * w2_v[OUT + k, :]
                g2_v[n, :] = acc * dn

        pltpu.sync_copy(g2_v, g2_hbm.at[sl])


def _sc_node2(s1, g1a, g1b, dis, b1, W2):
    return pl.kernel(
        _sc_node2_body,
        out_type=jax.ShapeDtypeStruct((N_PAD, OUT), _f32),
        mesh=_mesh,
        compiler_params=_sc_params,
        scratch_types=[
            pltpu.VMEM((BN, OUT), _f32),
            pltpu.VMEM((BN, OUT), _f32),
            pltpu.VMEM((BN, OUT), _f32),
            pltpu.VMEM((BN, OUT), _f32),
            pltpu.VMEM((BN,), _f32),
            pltpu.VMEM((HID, 16), _f32),
            pltpu.VMEM((HID, OUT), _f32),
            pltpu.VMEM((HID,), _f32),
            pltpu.VMEM((BN, OUT), _f32),
        ],
    )(s1, g1a, g1b, dis, b1, W2)


def _sc_node3_body(s2_hbm, g2_hbm, dis_hbm, b2_hbm, out_hbm,
                   sa_v, sb_v, g2_v, dis_v, b2_v, o_v):
    c = lax.axis_index("c")
    s = lax.axis_index("s")
    w = s * NC + c

    pltpu.sync_copy(b2_hbm, b2_v)
    b2r = b2_v[...]
    zero = jnp.zeros((16,), _f32)
    iota = lax.iota(_i32, 16)

    @pl.loop(w * NVB // NW, (w + 1) * NVB // NW)
    def _row(r):
        node0 = jnp.minimum(r * BN, N - BN)
        sl = pl.ds(node0, BN)
        pltpu.sync_copy(s2_hbm.at[0, sl], sa_v)
        pltpu.sync_copy(s2_hbm.at[1, sl], sb_v)
        pltpu.sync_copy(g2_hbm.at[sl], g2_v)
        pltpu.sync_copy(dis_hbm.at[sl], dis_v)

        @pl.loop(0, BN // 16)
        def _grp(t):
            base16 = t * 16
            dis16 = dis_v[pl.ds(base16, 16)]
            for j in range(16):
                n = base16 + j
                o_v[n, :] = jnp.maximum(
                    (sa_v[n, :] + sb_v[n, :] + g2_v[n, :]) * dis16[j] + b2r,
                    zero)

        pltpu.sync_copy(o_v, out_hbm.at[sl])


def _sc_node3(s2, g2, dis, b2):
    return pl.kernel(
        _sc_node3_body,
        out_type=jax.ShapeDtypeStruct((N, OUT), _f32),
        mesh=_mesh,
        compiler_params=_sc_params,
        scratch_types=[
            pltpu.VMEM((BN, OUT), _f32),
            pltpu.VMEM((BN, OUT), _f32),
            pltpu.VMEM((BN, OUT), _f32),
            pltpu.VMEM((BN,), _f32),
            pltpu.VMEM((OUT,), _f32),
            pltpu.VMEM((BN, OUT), _f32),
        ],
    )(s2, g2, dis, b2)


def kernel(x, edge_index, emb_a, emb_b, W1, b1, W2, b2):
    srcf = edge_index[0].astype(_i32)
    dst_r = edge_index[1].astype(_i32).reshape(E_ROWS, LANE)

    ta, tb = _tc_tables(emb_a, emb_b, W1)
    degp = _sc_deg(dst_r).reshape(NC, N_PAD)
    dis, g1a, g1b = _sc_node1(x, ta, tb, W1, degp)
    s1 = _sc_scatter1(srcf, dst_r, g1a, g1b)
    g2 = _sc_node2(s1, g1a, g1b, dis, b1, W2)
    s2 = _sc_scatter2(srcf, dst_r, g2)
    return _sc_node3(s2, g2, dis, b2)


# speedup vs baseline: 1.3007x; 1.0876x over previous
import functools

import jax
import jax.numpy as jnp
from jax import lax
from jax.experimental import pallas as pl
from jax.experimental.pallas import tpu as pltpu
from jax.experimental.pallas import tpu_sc as plsc

N = 50000
E = 1600000
OUT = 16
HID = 32
LANE = 128

NV = 391
BN = 512
NVB = 98
N_ROWS = 400
N_PAD = N_ROWS * LANE
E_ROWS = 12500
NC, NS = 2, 16
NW = NC * NS
NPT = N_PAD // NS
CH = 56
G = 7
NGRP = CH // G
NCHK = E_ROWS // CH
REM0 = NCHK * CH
REMR = E_ROWS - REM0

_mesh = plsc.VectorSubcoreMesh(core_axis_name="c", subcore_axis_name="s")
_f32 = jnp.float32
_i32 = jnp.int32
_sc_params = pltpu.CompilerParams(use_tc_tiling_on_sc=False,
                                  needs_layout_passes=False)


def _rsqrt16(d):
    xi = plsc.bitcast(d, _i32)
    y = plsc.bitcast(jnp.int32(0x5F3759DF) - (xi >> 1), _f32)
    for _ in range(3):
        y = y * (1.5 - 0.5 * d * y * y)
    return y


def _tc_tables_body(ea, eb, W1, ta_o, tb_o):
    ta_o[...] = jnp.dot(ea[...], W1[0:16, :], preferred_element_type=_f32)
    tb_o[...] = jnp.dot(eb[...], W1[16:32, :], preferred_element_type=_f32)


def _tc_tables(emb_a, emb_b, W1):
    return pl.pallas_call(
        _tc_tables_body,
        out_shape=[
            jax.ShapeDtypeStruct((1000, HID), _f32),
            jax.ShapeDtypeStruct((1000, HID), _f32),
        ],
    )(emb_a, emb_b, W1)


RPT = N_ROWS // 16


def _sc_deg_body(dst_hbm, z2_hbm, deg_hbm, idx_v, ridx_v, ld_v, deg_sp):
    c = lax.axis_index("c")
    s = lax.axis_index("s")
    ones16 = jnp.ones((16,), _f32)
    zeros16 = jnp.zeros((16,), _f32)
    iota = lax.iota(_i32, 16)

    @pl.loop(0, N_ROWS)
    def _z(r):
        for u in range(LANE // 16):
            ld_v[r, pl.ds(u * 16, 16)] = zeros16

    for k in range(RPT):
        ridx_v[k, :] = iota + k * 16

    pltpu.sync_copy(z2_hbm, deg_sp.at[pl.ds(s * RPT, RPT)])
    plsc.subcore_barrier()

    w = s * NC + c

    @pl.loop(w * NCHK // NW, (w + 1) * NCHK // NW)
    def _deg_chunk(k):
        pltpu.sync_copy(dst_hbm.at[pl.ds(k * CH, CH)], idx_v)

        @pl.loop(0, CH)
        def _deg_row(j):
            for u in range(LANE // 16):
                d16 = idx_v[j, pl.ds(u * 16, 16)]
                plsc.addupdate_scatter(ld_v, [d16 >> 7, d16 & 127], ones16)

    @pl.when(w == NW - 1)
    def _deg_rem():
        pltpu.sync_copy(dst_hbm.at[pl.ds(REM0, REMR)],
                        idx_v.at[pl.ds(0, REMR)])

        @pl.loop(0, REMR)
        def _deg_rem_row(j):
            for u in range(LANE // 16):
                d16 = idx_v[j, pl.ds(u * 16, 16)]
                plsc.addupdate_scatter(ld_v, [d16 >> 7, d16 & 127], ones16)

    @pl.loop(0, RPT)
    def _merge(k):
        pltpu.sync_copy(ld_v.at[pl.ds(k * 16, 16)],
                        deg_sp.at[ridx_v.at[k]], add=True)

    plsc.subcore_barrier()
    pltpu.sync_copy(deg_sp.at[pl.ds(s * RPT, RPT)],
                    deg_hbm.at[c, pl.ds(s * RPT, RPT)])


def _sc_deg(dst_r):
    z2 = jnp.zeros((RPT, LANE), _f32)
    return pl.kernel(
        _sc_deg_body,
        out_type=jax.ShapeDtypeStruct((NC, N_ROWS, LANE), _f32),
        mesh=_mesh,
        compiler_params=_sc_params,
        scratch_types=[
            pltpu.VMEM((CH, LANE), _i32),
            pltpu.VMEM((RPT, 16), _i32),
            pltpu.VMEM((N_ROWS, LANE), _f32),
            pltpu.VMEM_SHARED((N_ROWS, LANE), _f32),
        ],
    )(dst_r, z2)


def _sc_node1_body(x_hbm, ta_hbm, tb_hbm, W1_hbm, degp_hbm,
                   dis_hbm, g1a_hbm, g1b_hbm,
                   ta_v, tb_v, x_v, w1_v, deg_v, dis_v, ga_v, gb_v):
    c = lax.axis_index("c")
    s = lax.axis_index("s")
    w = s * NC + c

    pltpu.sync_copy(ta_hbm, ta_v)
    pltpu.sync_copy(tb_hbm, tb_v)
    pltpu.sync_copy(W1_hbm, w1_v)
    w1a = [w1_v[32 + k, pl.ds(0, 16)] for k in range(4)]
    w1b = [w1_v[32 + k, pl.ds(16, 16)] for k in range(4)]
    iota = lax.iota(_i32, 16)

    @pl.loop(w * NVB // NW, (w + 1) * NVB // NW)
    def _row(r):
        node0 = jnp.minimum(r * BN, N - BN)
        pltpu.sync_copy(x_hbm.at[pl.ds(node0, BN)], x_v)
        pltpu.sync_copy(degp_hbm.at[0, pl.ds(node0, BN)], deg_v.at[0])
        pltpu.sync_copy(degp_hbm.at[1, pl.ds(node0, BN)], deg_v.at[1])
        for v in range(BN // 16):
            sl = pl.ds(v * 16, 16)
            d = deg_v[0, sl] + deg_v[1, sl] + 1.0
            dis_v[sl] = _rsqrt16(d)
        pltpu.sync_copy(dis_v, dis_hbm.at[pl.ds(node0, BN)])

        @pl.loop(0, BN // 16)
        def _grp(t):
            base16 = t * 16
            nidx = iota + base16
            dis16 = dis_v[pl.ds(base16, 16)]
            ia16 = plsc.load_gather(
                x_v, [nidx, jnp.zeros((16,), _i32)]).astype(_i32)
            ib16 = plsc.load_gather(
                x_v, [nidx, jnp.full((16,), 1, _i32)]).astype(_i32)
            nums = [plsc.load_gather(x_v, [nidx, jnp.full((16,), 2 + k, _i32)])
                    for k in range(4)]
            for j in range(16):
                ian = ia16[j]
                ibn = ib16[j]
                ha = ta_v[ian, pl.ds(0, 16)] + tb_v[ibn, pl.ds(0, 16)]
                hb = ta_v[ian, pl.ds(16, 16)] + tb_v[ibn, pl.ds(16, 16)]
                for k in range(4):
                    nk = nums[k][j]
                    ha = ha + nk * w1a[k]
                    hb = hb + nk * w1b[k]
                dn = dis16[j]
                ga_v[base16 + j, :] = ha * dn
                gb_v[base16 + j, :] = hb * dn

        pltpu.sync_copy(ga_v, g1a_hbm.at[pl.ds(node0, BN)])
        pltpu.sync_copy(gb_v, g1b_hbm.at[pl.ds(node0, BN)])


def _sc_node1(x, ta, tb, W1, degp):
    return pl.kernel(
        _sc_node1_body,
        out_type=[
            jax.ShapeDtypeStruct((N_PAD,), _f32),
            jax.ShapeDtypeStruct((N_PAD, OUT), _f32),
            jax.ShapeDtypeStruct((N_PAD, OUT), _f32),
        ],
        mesh=_mesh,
        compiler_params=_sc_params,
        scratch_types=[
            pltpu.VMEM((1000, HID), _f32),
            pltpu.VMEM((1000, HID), _f32),
            pltpu.VMEM((BN, 6), _f32),
            pltpu.VMEM((36, HID), _f32),
            pltpu.VMEM((2, BN), _f32),
            pltpu.VMEM((BN,), _f32),
            pltpu.VMEM((BN, OUT), _f32),
            pltpu.VMEM((BN, OUT), _f32),
        ],
    )(x, ta, tb, W1, degp)


def _edge_pipeline(g_hbm, srcf_hbm, dst_hbm, acc_sp,
                   sidx_v, didx_v, rows_v, gsems, ssems, chunk_lo, chunk_hi):
    def FG(g, q):
        pltpu.async_copy(g_hbm.at[sidx_v.at[pl.ds(g * G * LANE, G * LANE)]],
                         rows_v.at[q], gsems[q])

    def WG(g, q):
        pltpu.make_async_copy(
            g_hbm.at[sidx_v.at[pl.ds(g * G * LANE, G * LANE)]],
            rows_v.at[q], gsems[q]).wait()

    def FS(g, q):
        @pl.loop(0, G)
        def _fs(b):
            pltpu.async_copy(rows_v.at[q, pl.ds(b * LANE, LANE)],
                             acc_sp.at[didx_v.at[g * G + b]], ssems[q],
                             add=True)

    def WS(g, q):
        pltpu.make_async_copy(rows_v.at[q],
                              acc_sp.at[pl.ds(0, G * LANE)],
                              ssems[q]).wait()

    @pl.loop(chunk_lo, chunk_hi)
    def _chunk(k):
        row0 = k * CH
        pltpu.sync_copy(srcf_hbm.at[pl.ds(row0 * LANE, CH * LANE)], sidx_v)
        pltpu.sync_copy(dst_hbm.at[pl.ds(row0, CH)], didx_v)
        for q in range(4):
            FG(q, q)
        for q in range(4):
            WG(q, q)
            FS(q, q)
        for q in range(4):
            WS(q, q)
            FG(4 + q, q)
        for q in range(4):
            WG(4 + q, q)
            FS(4 + q, q)
        for q in range(4):
            WS(4 + q, q)


def _edge_remainder(g_hbm, srcf_hbm, dst_hbm, acc_sp, sidx_v, didx_v,
                    rows_v, gsem):
    hw = REMR // 2
    pltpu.sync_copy(srcf_hbm.at[pl.ds(REM0 * LANE, REMR * LANE)],
                    sidx_v.at[pl.ds(0, REMR * LANE)])
    pltpu.sync_copy(dst_hbm.at[pl.ds(REM0, REMR)], didx_v.at[pl.ds(0, REMR)])
    for p in range(2):
        cp = pltpu.async_copy(
            g_hbm.at[sidx_v.at[pl.ds(p * hw * LANE, hw * LANE)]],
            rows_v.at[0, pl.ds(0, hw * LANE)], gsem)
        cp.wait()
        for b in range(hw):
            pltpu.sync_copy(rows_v.at[0, pl.ds(b * LANE, LANE)],
                            acc_sp.at[didx_v.at[p * hw + b]], add=True)


def _sc_scatter1_body(src_hbm, dst_hbm, ga_hbm, gb_hbm, z_hbm, out_hbm,
                      sidx_v, didx_v, rows_v, g0, g1, g2, g3,
                      s0, s1, s2, s3, acc_sp):
    c = lax.axis_index("c")
    s = lax.axis_index("s")

    pltpu.sync_copy(z_hbm, acc_sp.at[pl.ds(s * NPT, NPT)])
    plsc.subcore_barrier()

    args = (src_hbm, dst_hbm, acc_sp, sidx_v, didx_v, rows_v,
            (g0, g1, g2, g3), (s0, s1, s2, s3))

    lo = s * NCHK // NS
    hi = (s + 1) * NCHK // NS

    @pl.when(c == 0)
    def _half_a():
        _edge_pipeline(ga_hbm, *args, chunk_lo=lo, chunk_hi=hi)

        @pl.when(s == NS - 1)
        def _rem_a():
            _edge_remainder(ga_hbm, src_hbm, dst_hbm, acc_sp,
                            sidx_v, didx_v, rows_v, g0)

    @pl.when(c == 1)
    def _half_b():
        _edge_pipeline(gb_hbm, *args, chunk_lo=lo, chunk_hi=hi)

        @pl.when(s == NS - 1)
        def _rem_b():
            _edge_remainder(gb_hbm, src_hbm, dst_hbm, acc_sp,
                            sidx_v, didx_v, rows_v, g0)

    plsc.subcore_barrier()
    pltpu.sync_copy(acc_sp.at[pl.ds(s * NPT, NPT)],
                    out_hbm.at[c, pl.ds(s * NPT, NPT)])


def _sc_scatter1(srcf, dst_r, ga, gb):
    z = jnp.zeros((NPT, OUT), _f32)
    return pl.kernel(
        _sc_scatter1_body,
        out_type=jax.ShapeDtypeStruct((NC, N_PAD, OUT), _f32),
        mesh=_mesh,
        compiler_params=_sc_params,
        scratch_types=[
            pltpu.VMEM((CH * LANE,), _i32),
            pltpu.VMEM((CH, LANE), _i32),
            pltpu.VMEM((4, G * LANE, OUT), _f32),
            pltpu.SemaphoreType.DMA,
            pltpu.SemaphoreType.DMA,
            pltpu.SemaphoreType.DMA,
            pltpu.SemaphoreType.DMA,
            pltpu.SemaphoreType.DMA,
            pltpu.SemaphoreType.DMA,
            pltpu.SemaphoreType.DMA,
            pltpu.SemaphoreType.DMA,
            pltpu.VMEM_SHARED((N_PAD, OUT), _f32),
        ],
    )(srcf, dst_r, ga, gb, z)


def _sc_scatter2_body(src_hbm, dst_hbm, g_hbm, z_hbm, out_hbm,
                      sidx_v, didx_v, rows_v, g0, g1, g2, g3,
                      s0, s1, s2, s3, acc_sp):
    c = lax.axis_index("c")
    s = lax.axis_index("s")

    pltpu.sync_copy(z_hbm, acc_sp.at[pl.ds(s * NPT, NPT)])
    plsc.subcore_barrier()

    w = s * NC + c
    _edge_pipeline(g_hbm, src_hbm, dst_hbm, acc_sp, sidx_v, didx_v, rows_v,
                   (g0, g1, g2, g3), (s0, s1, s2, s3),
                   chunk_lo=w * NCHK // NW, chunk_hi=(w + 1) * NCHK // NW)

    @pl.when(w == NW - 1)
    def _rem():
        _edge_remainder(g_hbm, src_hbm, dst_hbm, acc_sp,
                        sidx_v, didx_v, rows_v, g0)

    plsc.subcore_barrier()
    pltpu.sync_copy(acc_sp.at[pl.ds(s * NPT, NPT)],
                    out_hbm.at[c, pl.ds(s * NPT, NPT)])


def _sc_scatter2(srcf, dst_r, g):
    z = jnp.zeros((NPT, OUT), _f32)
    return pl.kernel(
        _sc_scatter2_body,
        out_type=jax.ShapeDtypeStruct((NC, N_PAD, OUT), _f32),
        mesh=_mesh,
        compiler_params=_sc_params,
        scratch_types=[
            pltpu.VMEM((CH * LANE,), _i32),
            pltpu.VMEM((CH, LANE), _i32),
            pltpu.VMEM((4, G * LANE, OUT), _f32),
            pltpu.SemaphoreType.DMA,
            pltpu.SemaphoreType.DMA,
            pltpu.SemaphoreType.DMA,
            pltpu.SemaphoreType.DMA,
            pltpu.SemaphoreType.DMA,
            pltpu.SemaphoreType.DMA,
            pltpu.SemaphoreType.DMA,
            pltpu.SemaphoreType.DMA,
            pltpu.VMEM_SHARED((N_PAD, OUT), _f32),
        ],
    )(srcf, dst_r, g, z)


def _sc_node2_body(s1_hbm, g1a_hbm, g1b_hbm, dis_hbm, b1_hbm, W2_hbm,
                   g2_hbm,
                   sa_v, sb_v, ga_v, gb_v, dis_v, o1_v, w2_v, b1_v, g2_v):
    c = lax.axis_index("c")
    s = lax.axis_index("s")
    w = s * NC + c

    pltpu.sync_copy(W2_hbm, w2_v)
    pltpu.sync_copy(b1_hbm, b1_v)
    b1a = b1_v[pl.ds(0, 16)]
    b1b = b1_v[pl.ds(16, 16)]
    zero = jnp.zeros((16,), _f32)
    iota = lax.iota(_i32, 16)

    @pl.loop(w * NVB // NW, (w + 1) * NVB // NW)
    def _row(r):
        node0 = jnp.minimum(r * BN, N - BN)
        sl = pl.ds(node0, BN)
        pltpu.sync_copy(s1_hbm.at[0, sl], sa_v)
        pltpu.sync_copy(s1_hbm.at[1, sl], sb_v)
        pltpu.sync_copy(g1a_hbm.at[sl], ga_v)
        pltpu.sync_copy(g1b_hbm.at[sl], gb_v)
        pltpu.sync_copy(dis_hbm.at[sl], dis_v)

        @pl.loop(0, BN // 16)
        def _grp(t):
            base16 = t * 16
            dis16 = dis_v[pl.ds(base16, 16)]
            for j in range(16):
                n = base16 + j
                dn = dis16[j]
                o1a = jnp.maximum((sa_v[n, :] + ga_v[n, :]) * dn + b1a, zero)
                o1b = jnp.maximum((sb_v[n, :] + gb_v[n, :]) * dn + b1b, zero)
                acc = zero
                for k in range(OUT):
                    acc = acc + o1a[k] * w2_v[k, :]
                for k in range(OUT):
                    acc = acc + o1b[k] * w2_v[OUT + k, :]
                g2_v[n, :] = acc * dn

        pltpu.sync_copy(g2_v, g2_hbm.at[sl])


def _sc_node2(s1, g1a, g1b, dis, b1, W2):
    return pl.kernel(
        _sc_node2_body,
        out_type=jax.ShapeDtypeStruct((N_PAD, OUT), _f32),
        mesh=_mesh,
        compiler_params=_sc_params,
        scratch_types=[
            pltpu.VMEM((BN, OUT), _f32),
            pltpu.VMEM((BN, OUT), _f32),
            pltpu.VMEM((BN, OUT), _f32),
            pltpu.VMEM((BN, OUT), _f32),
            pltpu.VMEM((BN,), _f32),
            pltpu.VMEM((HID, 16), _f32),
            pltpu.VMEM((HID, OUT), _f32),
            pltpu.VMEM((HID,), _f32),
            pltpu.VMEM((BN, OUT), _f32),
        ],
    )(s1, g1a, g1b, dis, b1, W2)


def _sc_node3_body(s2_hbm, g2_hbm, dis_hbm, b2_hbm, out_hbm,
                   sa_v, sb_v, g2_v, dis_v, b2_v, o_v):
    c = lax.axis_index("c")
    s = lax.axis_index("s")
    w = s * NC + c

    pltpu.sync_copy(b2_hbm, b2_v)
    b2r = b2_v[...]
    zero = jnp.zeros((16,), _f32)
    iota = lax.iota(_i32, 16)

    @pl.loop(w * NVB // NW, (w + 1) * NVB // NW)
    def _row(r):
        node0 = jnp.minimum(r * BN, N - BN)
        sl = pl.ds(node0, BN)
        pltpu.sync_copy(s2_hbm.at[0, sl], sa_v)
        pltpu.sync_copy(s2_hbm.at[1, sl], sb_v)
        pltpu.sync_copy(g2_hbm.at[sl], g2_v)
        pltpu.sync_copy(dis_hbm.at[sl], dis_v)

        @pl.loop(0, BN // 16)
        def _grp(t):
            base16 = t * 16
            dis16 = dis_v[pl.ds(base16, 16)]
            for j in range(16):
                n = base16 + j
                o_v[n, :] = jnp.maximum(
                    (sa_v[n, :] + sb_v[n, :] + g2_v[n, :]) * dis16[j] + b2r,
                    zero)

        pltpu.sync_copy(o_v, out_hbm.at[sl])


def _sc_node3(s2, g2, dis, b2):
    return pl.kernel(
        _sc_node3_body,
        out_type=jax.ShapeDtypeStruct((N, OUT), _f32),
        mesh=_mesh,
        compiler_params=_sc_params,
        scratch_types=[
            pltpu.VMEM((BN, OUT), _f32),
            pltpu.VMEM((BN, OUT), _f32),
            pltpu.VMEM((BN, OUT), _f32),
            pltpu.VMEM((BN,), _f32),
            pltpu.VMEM((OUT,), _f32),
            pltpu.VMEM((BN, OUT), _f32),
        ],
    )(s2, g2, dis, b2)


def kernel(x, edge_index, emb_a, emb_b, W1, b1, W2, b2):
    srcf = edge_index[0].astype(_i32)
    dst_r = edge_index[1].astype(_i32).reshape(E_ROWS, LANE)

    ta, tb = _tc_tables(emb_a, emb_b, W1)
    degp = _sc_deg(dst_r).reshape(NC, N_PAD)
    dis, g1a, g1b = _sc_node1(x, ta, tb, W1, degp)
    s1 = _sc_scatter1(srcf, dst_r, g1a, g1b)
    g2 = _sc_node2(s1, g1a, g1b, dis, b1, W2)
    s2 = _sc_scatter2(srcf, dst_r, g2)
    return _sc_node3(s2, g2, dis, b2)
